# Initial kernel scaffold; baseline (speedup 1.0000x reference)
import functools

import jax
import jax.numpy as jnp
from jax import lax
from jax.experimental import pallas as pl
from jax.experimental.pallas import tpu as pltpu
from jax.experimental.pallas import tpu_sc as plsc

RADIUS2 = 4.0
HMIN = -1.0
HMAX = 1.0
S = 64
INVALID = 1e10
VALID_CUT = 1e9
KNOCK = 3e10


def _select_body(xyz_ref, c_ref, r_ref, idx_ref, dist_ref, *, n, pb):
    b = pl.program_id(0)
    x = xyz_ref[0, 0:1, :]
    y = xyz_ref[0, 1:2, :]
    z = xyz_ref[0, 2:3, :]
    cx = c_ref[0, :, 0:1]
    cy = c_ref[0, :, 1:2]
    cz = c_ref[0, :, 2:3]
    r = r_ref[0]
    dx = x - cx
    dy = y - cy
    dz = z - cz
    lx = dx * r[:, 0:1] + dy * r[:, 3:4] + dz * r[:, 6:7]
    ly = dx * r[:, 1:2] + dy * r[:, 4:5] + dz * r[:, 7:8]
    lz = dx * r[:, 2:3] + dy * r[:, 5:6] + dz * r[:, 8:9]
    r2 = ly * ly + lz * lz
    in_cyl = (lx >= HMIN) & (lx <= HMAX) & (r2 < RADIUS2)
    dist_ref[...] = jnp.where(in_cyl, r2, INVALID)

    iota_n = lax.broadcasted_iota(jnp.int32, (pb, n), 1)
    iota_s = lax.broadcasted_iota(jnp.int32, (pb, S), 1)

    def body(s, carry):
        idx_acc, first = carry
        d = dist_ref[...]
        m = jnp.min(d, axis=1, keepdims=True)
        am = jnp.min(jnp.where(d == m, iota_n, n), axis=1,
                     keepdims=True).astype(jnp.int32)
        first = jnp.where(s == 0, am, first)
        sel = jnp.where(m < VALID_CUT, am, first)
        idx_acc = jnp.where(iota_s == s, sel, idx_acc)
        dist_ref[...] = jnp.where(iota_n == am, KNOCK, d)
        return idx_acc, first

    idx0 = jnp.zeros((pb, S), jnp.int32)
    first0 = jnp.zeros((pb, 1), jnp.int32)
    idx_acc, _ = lax.fori_loop(0, S, body, (idx0, first0))
    idx_ref[0] = idx_acc + b * n


def _select(xyz_t, new_xyz, rot9, *, pb=8):
    bsz, _, n = xyz_t.shape
    p = new_xyz.shape[1]
    grid = (bsz, p // pb)
    return pl.pallas_call(
        functools.partial(_select_body, n=n, pb=pb),
        grid=grid,
        in_specs=[
            pl.BlockSpec((1, 3, n), lambda b, j: (b, 0, 0)),
            pl.BlockSpec((1, pb, 3), lambda b, j: (b, j, 0)),
            pl.BlockSpec((1, pb, 9), lambda b, j: (b, j, 0)),
        ],
        out_specs=pl.BlockSpec((1, pb, S), lambda b, j: (b, j, 0)),
        out_shape=jax.ShapeDtypeStruct((bsz, p, S), jnp.int32),
        scratch_shapes=[pltpu.VMEM((pb, n), jnp.float32)],
    )(xyz_t, new_xyz, rot9)


_NC = 2
_NS = 16
_NW = _NC * _NS


def _gather(ftab, xtab, gidx, *, chunk=128):
    rows = gidx.shape[0]
    c = ftab.shape[1]
    xw = xtab.shape[1]
    rpw = rows // _NW
    nch = rpw // chunk
    mesh = plsc.VectorSubcoreMesh(core_axis_name="c", subcore_axis_name="s")

    def body(ftab_hbm, xtab_hbm, idx_hbm, outf_hbm, outx_hbm,
             idx_v, rows_f, rows_x, semf, semx):
        wid = lax.axis_index("s") * _NC + lax.axis_index("c")
        base = wid * rpw
        pltpu.sync_copy(idx_hbm.at[pl.ds(base, rpw)], idx_v)

        def step(j, _):
            off = j * chunk
            cidx = idx_v.at[pl.ds(off, chunk)]
            cpf = pltpu.async_copy(ftab_hbm.at[cidx], rows_f, semf)
            cpx = pltpu.async_copy(xtab_hbm.at[cidx], rows_x, semx)
            cpf.wait()
            cpx.wait()
            pltpu.sync_copy(rows_f, outf_hbm.at[pl.ds(base + off, chunk)])
            pltpu.sync_copy(rows_x, outx_hbm.at[pl.ds(base + off, chunk)])
            return 0

        lax.fori_loop(0, nch, step, 0)

    fn = pl.kernel(
        body,
        out_type=(
            jax.ShapeDtypeStruct((rows, c), jnp.float32),
            jax.ShapeDtypeStruct((rows, xw), jnp.float32),
        ),
        mesh=mesh,
        scratch_types=[
            pltpu.VMEM((rpw,), jnp.int32),
            pltpu.VMEM((chunk, c), jnp.float32),
            pltpu.VMEM((chunk, xw), jnp.float32),
            pltpu.SemaphoreType.DMA,
            pltpu.SemaphoreType.DMA,
        ],
    )
    return fn(ftab, xtab, gidx)


def _fixup_body(f_ref, x_ref, c_ref, r_ref, out_ref, *, pg, cch):
    g = f_ref[0, 0]
    feats = jnp.transpose(g)
    xr = jnp.transpose(x_ref[0, 0])

    def expand(v):
        return jnp.reshape(jnp.broadcast_to(v[:, None], (pg, S)), (1, pg * S))

    nc = c_ref[0]
    rel = [xr[d:d + 1, :] - expand(nc[:, d]) for d in range(3)]
    r = r_ref[0]
    outs = []
    for d in range(3):
        acc = rel[0] * expand(r[:, 0, d])
        acc = acc + rel[1] * expand(r[:, 1, d])
        acc = acc + rel[2] * expand(r[:, 2, d])
        outs.append(acc)
    res = jnp.concatenate(outs + [feats], axis=0)
    out_ref[0] = jnp.reshape(res, (3 + cch, pg, S))


def _fixup(frows, xrows, new_xyz, rot, *, pg=8):
    bsz, p, _ = new_xyz.shape
    cch = frows.shape[-1]
    fr = frows.reshape(bsz, p // pg, pg * S, cch)
    xr = xrows.reshape(bsz, p // pg, pg * S, 16)
    grid = (bsz, p // pg)
    return pl.pallas_call(
        functools.partial(_fixup_body, pg=pg, cch=cch),
        grid=grid,
        in_specs=[
            pl.BlockSpec((1, 1, pg * S, cch), lambda b, j: (b, j, 0, 0)),
            pl.BlockSpec((1, 1, pg * S, 16), lambda b, j: (b, j, 0, 0)),
            pl.BlockSpec((1, pg, 3), lambda b, j: (b, j, 0)),
            pl.BlockSpec((1, pg, 3, 3), lambda b, j: (b, j, 0, 0)),
        ],
        out_specs=pl.BlockSpec((1, 3 + cch, pg, S), lambda b, j: (b, 0, j, 0)),
        out_shape=jax.ShapeDtypeStruct((bsz, 3 + cch, p, S), jnp.float32),
    )(fr, xr, new_xyz, rot)


def kernel(xyz, new_xyz, rot, features):
    bsz, n, _ = xyz.shape
    p = new_xyz.shape[1]
    c = features.shape[1]

    xyz_t = jnp.transpose(xyz, (0, 2, 1))
    rot9 = rot.reshape(bsz, p, 9)
    gidx = _select(xyz_t, new_xyz, rot9)

    ftab = jnp.transpose(features, (0, 2, 1)).reshape(bsz * n, c)
    xtab = jnp.pad(xyz, ((0, 0), (0, 0), (0, 13))).reshape(bsz * n, 16)
    frows, xrows = _gather(ftab, xtab, gidx.reshape(-1))

    return _fixup(frows, xrows, new_xyz, rot)


# speedup vs baseline: 1.8852x; 1.8852x over previous
import functools

import jax
import jax.numpy as jnp
from jax import lax
from jax.experimental import pallas as pl
from jax.experimental.pallas import tpu as pltpu
from jax.experimental.pallas import tpu_sc as plsc

RADIUS2 = 4.0
HMIN = -1.0
HMAX = 1.0
S = 64
INVALID = 1e10
VALID_CUT = 1e9
KNOCK = 3e10


def _select_body(xyz_ref, c_ref, r_ref, idx_ref, dist_ref, *, n, pb):
    b = pl.program_id(0)
    x = xyz_ref[0, 0:1, :]
    y = xyz_ref[0, 1:2, :]
    z = xyz_ref[0, 2:3, :]
    cx = c_ref[0, :, 0:1]
    cy = c_ref[0, :, 1:2]
    cz = c_ref[0, :, 2:3]
    r = r_ref[0]
    bf = lambda a: a.astype(jnp.bfloat16).astype(jnp.float32)
    dx = bf(x - cx)
    dy = bf(y - cy)
    dz = bf(z - cz)
    rb = bf(r)
    lx = dx * rb[:, 0:1] + dy * rb[:, 3:4] + dz * rb[:, 6:7]
    ly = dx * rb[:, 1:2] + dy * rb[:, 4:5] + dz * rb[:, 7:8]
    lz = dx * rb[:, 2:3] + dy * rb[:, 5:6] + dz * rb[:, 8:9]
    r2 = ly * ly + lz * lz
    in_cyl = (lx >= HMIN) & (lx <= HMAX) & (r2 < RADIUS2)
    dist_ref[...] = jnp.where(in_cyl, r2, INVALID)

    iota_n = lax.broadcasted_iota(jnp.int32, (pb, n), 1)
    iota_s = lax.broadcasted_iota(jnp.int32, (pb, S), 1)

    def body(s, carry):
        idx_acc, first = carry
        d = dist_ref[...]
        m = jnp.min(d, axis=1, keepdims=True)
        am = jnp.min(jnp.where(d == m, iota_n, n), axis=1,
                     keepdims=True).astype(jnp.int32)
        first = jnp.where(s == 0, am, first)
        sel = jnp.where(m < VALID_CUT, am, first)
        idx_acc = jnp.where(iota_s == s, sel, idx_acc)
        dist_ref[...] = jnp.where(iota_n == am, KNOCK, d)
        return idx_acc, first

    idx0 = jnp.zeros((pb, S), jnp.int32)
    first0 = jnp.zeros((pb, 1), jnp.int32)
    idx_acc, _ = lax.fori_loop(0, S, body, (idx0, first0))
    idx_ref[0] = idx_acc + b * n


def _select(xyz_t, new_xyz, rot9, *, pb=8):
    bsz, _, n = xyz_t.shape
    p = new_xyz.shape[1]
    grid = (bsz, p // pb)
    return pl.pallas_call(
        functools.partial(_select_body, n=n, pb=pb),
        grid=grid,
        in_specs=[
            pl.BlockSpec((1, 3, n), lambda b, j: (b, 0, 0)),
            pl.BlockSpec((1, pb, 3), lambda b, j: (b, j, 0)),
            pl.BlockSpec((1, pb, 9), lambda b, j: (b, j, 0)),
        ],
        out_specs=pl.BlockSpec((1, pb, S), lambda b, j: (b, j, 0)),
        out_shape=jax.ShapeDtypeStruct((bsz, p, S), jnp.int32),
        scratch_shapes=[pltpu.VMEM((pb, n), jnp.float32)],
    )(xyz_t, new_xyz, rot9)


_NC = 2
_NS = 16
_NW = _NC * _NS


def _gather(ftab, xtab, gidx, *, chunk=128):
    rows = gidx.shape[0]
    c = ftab.shape[1]
    xw = xtab.shape[1]
    rpw = rows // _NW
    nch = rpw // chunk
    idx3 = gidx.reshape(_NW, nch, chunk)
    mesh = plsc.VectorSubcoreMesh(core_axis_name="c", subcore_axis_name="s")

    def body(ftab_hbm, xtab_hbm, idx_hbm, outf_hbm, outx_hbm,
             idx_v, rows_f, rows_x, semf, semx):
        wid = lax.axis_index("s") * _NC + lax.axis_index("c")
        base = wid * rpw
        pltpu.sync_copy(idx_hbm.at[wid], idx_v)

        def step(j, _):
            cidx = idx_v.at[j]
            cpf = pltpu.async_copy(ftab_hbm.at[cidx], rows_f, semf)
            cpx = pltpu.async_copy(xtab_hbm.at[cidx], rows_x, semx)
            cpf.wait()
            cpx.wait()
            off = base + j * chunk
            pltpu.sync_copy(rows_f, outf_hbm.at[pl.ds(off, chunk)])
            pltpu.sync_copy(rows_x, outx_hbm.at[pl.ds(off, chunk)])
            return 0

        lax.fori_loop(0, nch, step, 0)

    fn = pl.kernel(
        body,
        out_type=(
            jax.ShapeDtypeStruct((rows, c), jnp.float32),
            jax.ShapeDtypeStruct((rows, xw), jnp.float32),
        ),
        mesh=mesh,
        scratch_types=[
            pltpu.VMEM((nch, chunk), jnp.int32),
            pltpu.VMEM((chunk, c), jnp.float32),
            pltpu.VMEM((chunk, xw), jnp.float32),
            pltpu.SemaphoreType.DMA,
            pltpu.SemaphoreType.DMA,
        ],
    )
    return fn(ftab, xtab, idx3)


def _fixup_body(f_ref, x_ref, c_ref, r_ref, out_ref, *, pg, cch):
    w = pg * S
    feats = jnp.transpose(f_ref[0, 0])
    xt = jnp.transpose(x_ref[0, 0])
    lane = lax.broadcasted_iota(jnp.int32, (1, w), 1)
    pidx = lane // S

    def expand(read):
        acc = jnp.full((1, w), read(0), jnp.float32)
        for p in range(1, pg):
            acc = jnp.where(pidx == p, read(p), acc)
        return acc

    bf = lambda a: a.astype(jnp.bfloat16).astype(jnp.float32)
    rel = [bf(xt[d:d + 1, :] - expand(lambda p, d=d: c_ref[0, p, d]))
           for d in range(3)]
    outs = []
    for d in range(3):
        acc = rel[0] * bf(expand(lambda p, d=d: r_ref[0, p, d]))
        acc = acc + rel[1] * bf(expand(lambda p, d=d: r_ref[0, p, 3 + d]))
        acc = acc + rel[2] * bf(expand(lambda p, d=d: r_ref[0, p, 6 + d]))
        outs.append(acc)
    res = jnp.concatenate(outs + [feats], axis=0)
    out_ref[0] = jnp.reshape(res, (3 + cch, pg, S))


def _fixup(frows, xrows, new_xyz, rot9, *, pg=8):
    bsz, p, _ = new_xyz.shape
    cch = frows.shape[-1]
    fr = frows.reshape(bsz, p // pg, pg * S, cch)
    xr = xrows.reshape(bsz, p // pg, pg * S, xrows.shape[-1])
    grid = (bsz, p // pg)
    out = pl.pallas_call(
        functools.partial(_fixup_body, pg=pg, cch=cch),
        grid=grid,
        in_specs=[
            pl.BlockSpec((1, 1, pg * S, cch), lambda b, j: (b, j, 0, 0)),
            pl.BlockSpec((1, 1, pg * S, xr.shape[-1]), lambda b, j: (b, j, 0, 0)),
            pl.BlockSpec((1, pg, 3), lambda b, j: (b, j, 0),
                         memory_space=pltpu.SMEM),
            pl.BlockSpec((1, pg, 9), lambda b, j: (b, j, 0),
                         memory_space=pltpu.SMEM),
        ],
        out_specs=pl.BlockSpec((1, 3 + cch, pg, S), lambda b, j: (b, 0, j, 0)),
        out_shape=jax.ShapeDtypeStruct((bsz, 3 + cch, p, S), jnp.float32),
    )(fr, xr, new_xyz, rot9)
    return out


def kernel(xyz, new_xyz, rot, features):
    bsz, n, _ = xyz.shape
    p = new_xyz.shape[1]
    c = features.shape[1]

    xyz_t = jnp.transpose(xyz, (0, 2, 1))
    rot9 = rot.reshape(bsz, p, 9)
    gidx = _select(xyz_t, new_xyz, rot9)

    ftab = jnp.transpose(features, (0, 2, 1)).reshape(bsz * n, c)
    xtab = jnp.pad(xyz, ((0, 0), (0, 0), (0, 125))).reshape(bsz * n, 128)
    frows, xrows = _gather(ftab, xtab, gidx.reshape(-1))

    return _fixup(frows, xrows, new_xyz, rot9)


# speedup vs baseline: 6.1185x; 3.2455x over previous
import functools

import numpy as np
import jax
import jax.numpy as jnp
from jax import lax
from jax.experimental import pallas as pl
from jax.experimental.pallas import tpu as pltpu
from jax.experimental.pallas import tpu_sc as plsc

RADIUS2 = 4.0
HMIN = -1.0
HMAX = 1.0
S = 64
INVALID = 1e10
VALID_CUT = 1e9
PAD = 1e30
CAP = 256
HI_K = int(np.float32(INVALID).view(np.int32)) // 65536 + 1
XW = 8

_NC = 2
_NS = 16
_NW = _NC * _NS


def _distthr_body(xyz_ref, c_ref, r_ref, dist_ref, thr_ref, *, n, pb):
    x = xyz_ref[0, 0:1, :]
    y = xyz_ref[0, 1:2, :]
    z = xyz_ref[0, 2:3, :]
    cx = c_ref[0, :, 0:1]
    cy = c_ref[0, :, 1:2]
    cz = c_ref[0, :, 2:3]
    r = r_ref[0]
    bf = lambda a: a.astype(jnp.bfloat16).astype(jnp.float32)
    dx = bf(x - cx)
    dy = bf(y - cy)
    dz = bf(z - cz)
    rb = bf(r)
    lx = dx * rb[:, 0:1] + dy * rb[:, 3:4] + dz * rb[:, 6:7]
    ly = dx * rb[:, 1:2] + dy * rb[:, 4:5] + dz * rb[:, 7:8]
    lz = dx * rb[:, 2:3] + dy * rb[:, 5:6] + dz * rb[:, 8:9]
    r2 = ly * ly + lz * lz
    in_cyl = (lx >= HMIN) & (lx <= HMAX) & (r2 < RADIUS2)
    d = jnp.where(in_cyl, r2, INVALID)
    dist_ref[...] = jnp.reshape(d, (pb, 1, n))

    lo0 = jnp.zeros((pb, 1), jnp.int32)
    hi0 = jnp.full((pb, 1), HI_K, jnp.int32)

    def it(_, carry):
        lo, hi = carry
        mid = (lo + hi) >> 1
        midf = lax.bitcast_convert_type(mid << 16, jnp.float32)
        cnt = jnp.sum(jnp.where(d <= midf, 1.0, 0.0), axis=1, keepdims=True)
        ge = cnt >= float(S)
        return jnp.where(ge, lo, mid), jnp.where(ge, mid, hi)

    _, hi = lax.fori_loop(0, 15, it, (lo0, hi0))
    t = lax.bitcast_convert_type(hi << 16, jnp.float32)
    thr_ref[...] = jnp.reshape(jnp.broadcast_to(t, (pb, 16)), (pb, 1, 16))


def _distthr(xyz_t, new_xyz, rot9, *, pb=8):
    bsz, _, n = xyz_t.shape
    p = new_xyz.shape[1]
    grid = (bsz, p // pb)
    return pl.pallas_call(
        functools.partial(_distthr_body, n=n, pb=pb),
        grid=grid,
        in_specs=[
            pl.BlockSpec((1, 3, n), lambda b, j: (b, 0, 0)),
            pl.BlockSpec((1, pb, 3), lambda b, j: (b, j, 0)),
            pl.BlockSpec((1, pb, 9), lambda b, j: (b, j, 0)),
        ],
        out_specs=[
            pl.BlockSpec((pb, 1, n), lambda b, j: (b * (p // pb) + j, 0, 0)),
            pl.BlockSpec((pb, 1, 16), lambda b, j: (b * (p // pb) + j, 0, 0)),
        ],
        out_shape=[
            jax.ShapeDtypeStruct((bsz * p, 1, n), jnp.float32),
            jax.ShapeDtypeStruct((bsz * p, 1, 16), jnp.float32),
        ],
    )(xyz_t, new_xyz, rot9)


def _kminmax(ak, ai, bk, bi):
    sel = ak <= bk
    return (jnp.where(sel, ak, bk), jnp.where(sel, ai, bi),
            jnp.where(sel, bk, ak), jnp.where(sel, bi, ai))


def _bitonic_fix(run):
    n = len(run)
    if n == 1:
        k, i = run[0]
        kk, ii = plsc.sort_key_val(k, i)
        return [(kk, ii)]
    h = n // 2
    lo, hi = [], []
    for j in range(h):
        lk, li, hk, hi_i = _kminmax(run[j][0], run[j][1],
                                    run[j + h][0], run[j + h][1])
        lo.append((lk, li))
        hi.append((hk, hi_i))
    return _bitonic_fix(lo) + _bitonic_fix(hi)


def _merge(a, b, trunc=False):
    n = len(a)
    bp = [(lax.rev(b[n - 1 - j][0], (0,)), lax.rev(b[n - 1 - j][1], (0,)))
          for j in range(n)]
    lo, hi = [], []
    for j in range(n):
        lk, li, hk, hi_i = _kminmax(a[j][0], a[j][1], bp[j][0], bp[j][1])
        lo.append((lk, li))
        hi.append((hk, hi_i))
    if trunc:
        return _bitonic_fix(lo)
    return _bitonic_fix(lo) + _bitonic_fix(hi)


def _sort_lowest64(pairs):
    runs = [[plsc.sort_key_val(k, i)] for k, i in pairs]
    runs = [_merge(runs[2 * j], runs[2 * j + 1]) for j in range(8)]
    runs = [_merge(runs[2 * j], runs[2 * j + 1]) for j in range(4)]
    runs = [_merge(runs[2 * j], runs[2 * j + 1], trunc=True) for j in range(2)]
    return _merge(runs[0], runs[1], trunc=True)


def _extract(dist, thr, xyz_flat, ftab, *, n, c, bsz):
    rows = dist.shape[0]
    rpw = rows // _NW
    mesh = plsc.VectorSubcoreMesh(core_axis_name="c", subcore_axis_name="s")
    nchunk = n // 16

    def body(dist_hbm, thr_hbm, xyzf_hbm, ftab_hbm, fout_hbm, xout_hbm,
             x0_v, x1_v, x2_v, drow_v, tv_v, svv_v, svi_v, gb_v, fb_v, xo_v,
             semf):
        wid = lax.axis_index("s") * _NC + lax.axis_index("c")
        b = wid // (_NW // bsz)
        pltpu.sync_copy(xyzf_hbm.at[pl.ds((b * 3 + 0) * n, n)], x0_v)
        pltpu.sync_copy(xyzf_hbm.at[pl.ds((b * 3 + 1) * n, n)], x1_v)
        pltpu.sync_copy(xyzf_hbm.at[pl.ds((b * 3 + 2) * n, n)], x2_v)
        iota = lax.iota(jnp.int32, 16)
        cutv = jnp.full((16,), VALID_CUT, jnp.float32)

        def row_body(i, _):
            row = wid * rpw + i
            pltpu.sync_copy(dist_hbm.at[row], drow_v)
            pltpu.sync_copy(thr_hbm.at[row], tv_v)
            tv = tv_v[0]
            padk = jnp.full((16,), PAD, jnp.float32)
            padi = jnp.zeros((16,), jnp.int32)
            for j in range(CAP // 16 + 1):
                svv_v[pl.ds(j * 16, 16)] = padk
                svi_v[pl.ds(j * 16, 16)] = padi

            def scan(cc, off):
                v = drow_v[0, pl.ds(cc * 16, 16)]
                m = (v <= tv) & (v < cutv)
                mi = m.astype(jnp.int32)
                pos = off + plsc.cumsum(mi) - mi
                plsc.store_scatter(svv_v, [pos], v, mask=m)
                plsc.store_scatter(svi_v, [pos], iota + cc * 16, mask=m)
                return jnp.minimum(off + jnp.sum(mi), CAP)

            lax.fori_loop(0, nchunk, scan, jnp.int32(0))

            pairs = [(svv_v[pl.ds(j * 16, 16)], svi_v[pl.ds(j * 16, 16)])
                     for j in range(CAP // 16)]
            low = _sort_lowest64(pairs)
            first = jnp.max(jnp.where(iota == 0, low[0][1], 0))
            fsplat = jnp.full((16,), first, jnp.int32)
            sel = [jnp.where(k < cutv, ii, fsplat) for k, ii in low]
            for j in range(4):
                gb_v[pl.ds(j * 16, 16)] = sel[j] + b * n
            cp = pltpu.async_copy(ftab_hbm.at[gb_v], fb_v, semf)
            zero16 = jnp.zeros((16,), jnp.int32)
            one16 = jnp.full((16,), 1, jnp.int32)
            two16 = jnp.full((16,), 2, jnp.int32)
            for j in range(4):
                rows16 = iota + j * 16
                gx = plsc.load_gather(x0_v, [sel[j]])
                gy = plsc.load_gather(x1_v, [sel[j]])
                gz = plsc.load_gather(x2_v, [sel[j]])
                plsc.store_scatter(xo_v, [rows16, zero16], gx)
                plsc.store_scatter(xo_v, [rows16, one16], gy)
                plsc.store_scatter(xo_v, [rows16, two16], gz)
            cp.wait()
            pltpu.sync_copy(fb_v, fout_hbm.at[pl.ds(row * S, S)])
            pltpu.sync_copy(xo_v, xout_hbm.at[pl.ds(row * S, S)])
            return 0

        lax.fori_loop(0, rpw, row_body, 0)

    fn = pl.kernel(
        body,
        out_type=(
            jax.ShapeDtypeStruct((rows * S, c), jnp.float32),
            jax.ShapeDtypeStruct((rows * S, XW), jnp.float32),
        ),
        mesh=mesh,
        compiler_params=pltpu.CompilerParams(needs_layout_passes=False),
        scratch_types=[
            pltpu.VMEM((n,), jnp.float32),
            pltpu.VMEM((n,), jnp.float32),
            pltpu.VMEM((n,), jnp.float32),
            pltpu.VMEM((1, n), jnp.float32),
            pltpu.VMEM((1, 16), jnp.float32),
            pltpu.VMEM((CAP + 16,), jnp.float32),
            pltpu.VMEM((CAP + 16,), jnp.int32),
            pltpu.VMEM((S,), jnp.int32),
            pltpu.VMEM((S, c), jnp.float32),
            pltpu.VMEM((S, XW), jnp.float32),
            pltpu.SemaphoreType.DMA,
        ],
    )
    return fn(dist, thr, xyz_flat, ftab)


def _fixup_body(f_ref, x_ref, c_ref, r_ref, out_ref, *, pg, cch):
    w = pg * S
    feats = jnp.transpose(f_ref[0, 0])
    xt = jnp.transpose(x_ref[0, 0])
    lane = lax.broadcasted_iota(jnp.int32, (1, w), 1)
    pidx = lane // S

    def expand(read):
        acc = jnp.full((1, w), read(0), jnp.float32)
        for p in range(1, pg):
            acc = jnp.where(pidx == p, read(p), acc)
        return acc

    bf = lambda a: a.astype(jnp.bfloat16).astype(jnp.float32)
    rel = [bf(xt[d:d + 1, :] - expand(lambda p, d=d: c_ref[0, p, d]))
           for d in range(3)]
    outs = []
    for d in range(3):
        acc = rel[0] * bf(expand(lambda p, d=d: r_ref[0, p, d]))
        acc = acc + rel[1] * bf(expand(lambda p, d=d: r_ref[0, p, 3 + d]))
        acc = acc + rel[2] * bf(expand(lambda p, d=d: r_ref[0, p, 6 + d]))
        outs.append(acc)
    res = jnp.concatenate(outs + [feats], axis=0)
    out_ref[0] = jnp.reshape(res, (3 + cch, pg, S))


def _fixup(frows, xrows, new_xyz, rot9, *, pg=8):
    bsz, p, _ = new_xyz.shape
    cch = frows.shape[-1]
    fr = frows.reshape(bsz, p // pg, pg * S, cch)
    xr = xrows.reshape(bsz, p // pg, pg * S, xrows.shape[-1])
    grid = (bsz, p // pg)
    out = pl.pallas_call(
        functools.partial(_fixup_body, pg=pg, cch=cch),
        grid=grid,
        in_specs=[
            pl.BlockSpec((1, 1, pg * S, cch), lambda b, j: (b, j, 0, 0)),
            pl.BlockSpec((1, 1, pg * S, xr.shape[-1]), lambda b, j: (b, j, 0, 0)),
            pl.BlockSpec((1, pg, 3), lambda b, j: (b, j, 0),
                         memory_space=pltpu.SMEM),
            pl.BlockSpec((1, pg, 9), lambda b, j: (b, j, 0),
                         memory_space=pltpu.SMEM),
        ],
        out_specs=pl.BlockSpec((1, 3 + cch, pg, S), lambda b, j: (b, 0, j, 0)),
        out_shape=jax.ShapeDtypeStruct((bsz, 3 + cch, p, S), jnp.float32),
    )(fr, xr, new_xyz, rot9)
    return out


def kernel(xyz, new_xyz, rot, features):
    bsz, n, _ = xyz.shape
    p = new_xyz.shape[1]
    c = features.shape[1]

    xyz_t = jnp.transpose(xyz, (0, 2, 1))
    rot9 = rot.reshape(bsz, p, 9)
    dist, thr = _distthr(xyz_t, new_xyz, rot9)

    ftab = jnp.transpose(features, (0, 2, 1)).reshape(bsz * n, c)
    frows, xrows = _extract(dist, thr, xyz_t.reshape(-1), ftab,
                            n=n, c=c, bsz=bsz)

    return _fixup(frows, xrows, new_xyz, rot9)


# speedup vs baseline: 7.4638x; 1.2199x over previous
import functools

import numpy as np
import jax
import jax.numpy as jnp
from jax import lax
from jax.experimental import pallas as pl
from jax.experimental.pallas import tpu as pltpu
from jax.experimental.pallas import tpu_sc as plsc

RADIUS2 = 4.0
HMIN = -1.0
HMAX = 1.0
S = 64
INVALID = 1e10
VALID_CUT = 1e9
PAD = 1e30
CAP = 256
HI_K = int(np.float32(INVALID).view(np.int32)) // 65536 + 1
XW = 8

_NC = 2
_NS = 16
_NW = _NC * _NS


def _distthr_body(xyz_ref, c_ref, r_ref, dist_ref, thr_ref, *, n, pb):
    x = xyz_ref[0, 0:1, :]
    y = xyz_ref[0, 1:2, :]
    z = xyz_ref[0, 2:3, :]
    cx = c_ref[0, :, 0:1]
    cy = c_ref[0, :, 1:2]
    cz = c_ref[0, :, 2:3]
    r = r_ref[0]
    bf = lambda a: a.astype(jnp.bfloat16).astype(jnp.float32)
    dx = bf(x - cx)
    dy = bf(y - cy)
    dz = bf(z - cz)
    rb = bf(r)
    lx = dx * rb[:, 0:1] + dy * rb[:, 3:4] + dz * rb[:, 6:7]
    ly = dx * rb[:, 1:2] + dy * rb[:, 4:5] + dz * rb[:, 7:8]
    lz = dx * rb[:, 2:3] + dy * rb[:, 5:6] + dz * rb[:, 8:9]
    r2 = ly * ly + lz * lz
    in_cyl = (lx >= HMIN) & (lx <= HMAX) & (r2 < RADIUS2)
    d = jnp.where(in_cyl, r2, INVALID)
    dist_ref[...] = jnp.reshape(d, (pb, 1, n))

    lo0 = jnp.zeros((pb, 1), jnp.int32)
    hi0 = jnp.full((pb, 1), HI_K, jnp.int32)

    def it(_, carry):
        lo, hi = carry
        mid = (lo + hi) >> 1
        midf = lax.bitcast_convert_type(mid << 16, jnp.float32)
        cnt = jnp.sum(jnp.where(d <= midf, 1.0, 0.0), axis=1, keepdims=True)
        ge = cnt >= float(S)
        return jnp.where(ge, lo, mid), jnp.where(ge, mid, hi)

    _, hi = lax.fori_loop(0, 15, it, (lo0, hi0))
    t = lax.bitcast_convert_type(hi << 16, jnp.float32)
    thr_ref[...] = jnp.reshape(jnp.broadcast_to(t, (pb, 16)), (pb, 1, 16))


def _distthr(xyz_t, new_xyz, rot9, *, pb=8):
    bsz, _, n = xyz_t.shape
    p = new_xyz.shape[1]
    grid = (bsz, p // pb)
    return pl.pallas_call(
        functools.partial(_distthr_body, n=n, pb=pb),
        grid=grid,
        in_specs=[
            pl.BlockSpec((1, 3, n), lambda b, j: (b, 0, 0)),
            pl.BlockSpec((1, pb, 3), lambda b, j: (b, j, 0)),
            pl.BlockSpec((1, pb, 9), lambda b, j: (b, j, 0)),
        ],
        out_specs=[
            pl.BlockSpec((pb, 1, n), lambda b, j: (b * (p // pb) + j, 0, 0)),
            pl.BlockSpec((pb, 1, 16), lambda b, j: (b * (p // pb) + j, 0, 0)),
        ],
        out_shape=[
            jax.ShapeDtypeStruct((bsz * p, 1, n), jnp.float32),
            jax.ShapeDtypeStruct((bsz * p, 1, 16), jnp.float32),
        ],
    )(xyz_t, new_xyz, rot9)


def _kminmax(ak, ai, bk, bi):
    sel = ak <= bk
    return (jnp.where(sel, ak, bk), jnp.where(sel, ai, bi),
            jnp.where(sel, bk, ak), jnp.where(sel, bi, ai))


def _bitonic_fix(run):
    n = len(run)
    if n == 1:
        k, i = run[0]
        kk, ii = plsc.sort_key_val(k, i)
        return [(kk, ii)]
    h = n // 2
    lo, hi = [], []
    for j in range(h):
        lk, li, hk, hi_i = _kminmax(run[j][0], run[j][1],
                                    run[j + h][0], run[j + h][1])
        lo.append((lk, li))
        hi.append((hk, hi_i))
    return _bitonic_fix(lo) + _bitonic_fix(hi)


def _merge(a, b, trunc=False):
    n = len(a)
    bp = [(lax.rev(b[n - 1 - j][0], (0,)), lax.rev(b[n - 1 - j][1], (0,)))
          for j in range(n)]
    lo, hi = [], []
    for j in range(n):
        lk, li, hk, hi_i = _kminmax(a[j][0], a[j][1], bp[j][0], bp[j][1])
        lo.append((lk, li))
        hi.append((hk, hi_i))
    if trunc:
        return _bitonic_fix(lo)
    return _bitonic_fix(lo) + _bitonic_fix(hi)


def _sort_lowest64(pairs):
    runs = [[plsc.sort_key_val(k, i)] for k, i in pairs]
    runs = [_merge(runs[2 * j], runs[2 * j + 1]) for j in range(8)]
    runs = [_merge(runs[2 * j], runs[2 * j + 1]) for j in range(4)]
    runs = [_merge(runs[2 * j], runs[2 * j + 1], trunc=True) for j in range(2)]
    return _merge(runs[0], runs[1], trunc=True)


def _extract(dist, thr, xyz_flat, ftab, *, n, c, bsz):
    rows = dist.shape[0]
    rpw = rows // _NW
    mesh = plsc.VectorSubcoreMesh(core_axis_name="c", subcore_axis_name="s")
    nchunk = n // 16

    def body(dist_hbm, thr_hbm, xyzf_hbm, ftab_hbm, fout_hbm, xout_hbm,
             x0_v, x1_v, x2_v, drow_v, tv_v, svv_v, svi_v, gb_v, fb_v, xo_v,
             semf):
        wid = lax.axis_index("s") * _NC + lax.axis_index("c")
        b = wid // (_NW // bsz)
        pltpu.sync_copy(xyzf_hbm.at[pl.ds((b * 3 + 0) * n, n)], x0_v)
        pltpu.sync_copy(xyzf_hbm.at[pl.ds((b * 3 + 1) * n, n)], x1_v)
        pltpu.sync_copy(xyzf_hbm.at[pl.ds((b * 3 + 2) * n, n)], x2_v)
        iota = lax.iota(jnp.int32, 16)
        cutv = jnp.full((16,), VALID_CUT, jnp.float32)

        def row_body(i, _):
            row = wid * rpw + i
            pltpu.sync_copy(dist_hbm.at[row], drow_v)
            pltpu.sync_copy(thr_hbm.at[row], tv_v)
            tv = tv_v[0]
            padk = jnp.full((16,), PAD, jnp.float32)
            padi = jnp.zeros((16,), jnp.int32)
            for j in range(CAP // 16 + 1):
                svv_v[pl.ds(j * 16, 16)] = padk
                svi_v[pl.ds(j * 16, 16)] = padi

            capv = jnp.full((16,), CAP, jnp.int32)

            def scan(cc, offv):
                v = drow_v[0, pl.ds(cc * 16, 16)]
                m = (v <= tv) & (v < cutv)
                mi = m.astype(jnp.int32)
                pos = offv + plsc.cumsum(mi) - mi
                plsc.store_scatter(svv_v, [pos], v, mask=m)
                plsc.store_scatter(svi_v, [pos], iota + cc * 16, mask=m)
                cnt = plsc.all_reduce_population_count(m)
                return jnp.minimum(offv + cnt, capv)

            lax.fori_loop(0, nchunk, scan, jnp.zeros((16,), jnp.int32))

            pairs = [(svv_v[pl.ds(j * 16, 16)], svi_v[pl.ds(j * 16, 16)])
                     for j in range(CAP // 16)]
            low = _sort_lowest64(pairs)
            first = jnp.max(jnp.where(iota == 0, low[0][1], 0))
            fsplat = jnp.full((16,), first, jnp.int32)
            sel = [jnp.where(k < cutv, ii, fsplat) for k, ii in low]
            for j in range(4):
                gb_v[pl.ds(j * 16, 16)] = sel[j] + b * n
            cp = pltpu.async_copy(ftab_hbm.at[gb_v], fb_v, semf)
            zero16 = jnp.zeros((16,), jnp.int32)
            one16 = jnp.full((16,), 1, jnp.int32)
            two16 = jnp.full((16,), 2, jnp.int32)
            for j in range(4):
                rows16 = iota + j * 16
                gx = plsc.load_gather(x0_v, [sel[j]])
                gy = plsc.load_gather(x1_v, [sel[j]])
                gz = plsc.load_gather(x2_v, [sel[j]])
                plsc.store_scatter(xo_v, [rows16, zero16], gx)
                plsc.store_scatter(xo_v, [rows16, one16], gy)
                plsc.store_scatter(xo_v, [rows16, two16], gz)
            cp.wait()
            pltpu.sync_copy(fb_v, fout_hbm.at[pl.ds(row * S, S)])
            pltpu.sync_copy(xo_v, xout_hbm.at[pl.ds(row * S, S)])
            return 0

        lax.fori_loop(0, rpw, row_body, 0)

    fn = pl.kernel(
        body,
        out_type=(
            jax.ShapeDtypeStruct((rows * S, c), jnp.float32),
            jax.ShapeDtypeStruct((rows * S, XW), jnp.float32),
        ),
        mesh=mesh,
        compiler_params=pltpu.CompilerParams(needs_layout_passes=False),
        scratch_types=[
            pltpu.VMEM((n,), jnp.float32),
            pltpu.VMEM((n,), jnp.float32),
            pltpu.VMEM((n,), jnp.float32),
            pltpu.VMEM((1, n), jnp.float32),
            pltpu.VMEM((1, 16), jnp.float32),
            pltpu.VMEM((CAP + 16,), jnp.float32),
            pltpu.VMEM((CAP + 16,), jnp.int32),
            pltpu.VMEM((S,), jnp.int32),
            pltpu.VMEM((S, c), jnp.float32),
            pltpu.VMEM((S, XW), jnp.float32),
            pltpu.SemaphoreType.DMA,
        ],
    )
    return fn(dist, thr, xyz_flat, ftab)


def _fixup_body(f_ref, x_ref, c_ref, r_ref, out_ref, *, pg, cch):
    w = pg * S
    feats = jnp.transpose(f_ref[0, 0])
    xt = jnp.transpose(x_ref[0, 0])
    lane = lax.broadcasted_iota(jnp.int32, (1, w), 1)
    pidx = lane // S

    def expand(read):
        acc = jnp.full((1, w), read(0), jnp.float32)
        for p in range(1, pg):
            acc = jnp.where(pidx == p, read(p), acc)
        return acc

    bf = lambda a: a.astype(jnp.bfloat16).astype(jnp.float32)
    rel = [bf(xt[d:d + 1, :] - expand(lambda p, d=d: c_ref[0, p, d]))
           for d in range(3)]
    outs = []
    for d in range(3):
        acc = rel[0] * bf(expand(lambda p, d=d: r_ref[0, p, d]))
        acc = acc + rel[1] * bf(expand(lambda p, d=d: r_ref[0, p, 3 + d]))
        acc = acc + rel[2] * bf(expand(lambda p, d=d: r_ref[0, p, 6 + d]))
        outs.append(acc)
    res = jnp.concatenate(outs + [feats], axis=0)
    out_ref[0] = jnp.reshape(res, (3 + cch, pg, S))


def _fixup(frows, xrows, new_xyz, rot9, *, pg=8):
    bsz, p, _ = new_xyz.shape
    cch = frows.shape[-1]
    fr = frows.reshape(bsz, p // pg, pg * S, cch)
    xr = xrows.reshape(bsz, p // pg, pg * S, xrows.shape[-1])
    grid = (bsz, p // pg)
    out = pl.pallas_call(
        functools.partial(_fixup_body, pg=pg, cch=cch),
        grid=grid,
        in_specs=[
            pl.BlockSpec((1, 1, pg * S, cch), lambda b, j: (b, j, 0, 0)),
            pl.BlockSpec((1, 1, pg * S, xr.shape[-1]), lambda b, j: (b, j, 0, 0)),
            pl.BlockSpec((1, pg, 3), lambda b, j: (b, j, 0),
                         memory_space=pltpu.SMEM),
            pl.BlockSpec((1, pg, 9), lambda b, j: (b, j, 0),
                         memory_space=pltpu.SMEM),
        ],
        out_specs=pl.BlockSpec((1, 3 + cch, pg, S), lambda b, j: (b, 0, j, 0)),
        out_shape=jax.ShapeDtypeStruct((bsz, 3 + cch, p, S), jnp.float32),
    )(fr, xr, new_xyz, rot9)
    return out


def kernel(xyz, new_xyz, rot, features):
    bsz, n, _ = xyz.shape
    p = new_xyz.shape[1]
    c = features.shape[1]

    xyz_t = jnp.transpose(xyz, (0, 2, 1))
    rot9 = rot.reshape(bsz, p, 9)
    dist, thr = _distthr(xyz_t, new_xyz, rot9)

    ftab = jnp.transpose(features, (0, 2, 1)).reshape(bsz * n, c)
    frows, xrows = _extract(dist, thr, xyz_t.reshape(-1), ftab,
                            n=n, c=c, bsz=bsz)

    return _fixup(frows, xrows, new_xyz, rot9)


# speedup vs baseline: 8.2541x; 1.1059x over previous
import functools

import numpy as np
import jax
import jax.numpy as jnp
from jax import lax
from jax.experimental import pallas as pl
from jax.experimental.pallas import tpu as pltpu
from jax.experimental.pallas import tpu_sc as plsc

RADIUS2 = 4.0
HMIN = -1.0
HMAX = 1.0
S = 64
INVALID = 1e10
VALID_CUT = 1e9
PAD = 1e30
CAP = 128
HI_K = int(np.float32(INVALID).view(np.int32)) // 65536 + 1
XW = 8

_NC = 2
_NS = 16
_NW = _NC * _NS


def _distthr_body(xyz_ref, c_ref, r_ref, dist_ref, *, n, pb):
    x = xyz_ref[0, 0:1, :]
    y = xyz_ref[0, 1:2, :]
    z = xyz_ref[0, 2:3, :]
    cx = c_ref[0, :, 0:1]
    cy = c_ref[0, :, 1:2]
    cz = c_ref[0, :, 2:3]
    r = r_ref[0]
    bf = lambda a: a.astype(jnp.bfloat16).astype(jnp.float32)
    dx = bf(x - cx)
    dy = bf(y - cy)
    dz = bf(z - cz)
    rb = bf(r)
    lx = dx * rb[:, 0:1] + dy * rb[:, 3:4] + dz * rb[:, 6:7]
    ly = dx * rb[:, 1:2] + dy * rb[:, 4:5] + dz * rb[:, 7:8]
    lz = dx * rb[:, 2:3] + dy * rb[:, 5:6] + dz * rb[:, 8:9]
    r2 = ly * ly + lz * lz
    in_cyl = (lx >= HMIN) & (lx <= HMAX) & (r2 < RADIUS2)
    d = jnp.where(in_cyl, r2, INVALID)

    key16 = (lax.bitcast_convert_type(d, jnp.int32) >> 16).astype(jnp.int16)
    lo0 = jnp.full((pb, 1), -1, jnp.int32)
    hi0 = jnp.full((pb, 1), HI_K - 1, jnp.int32)

    nsub = n // 16

    def it(_, carry):
        lo, hi = carry
        mid = (lo + hi) >> 1
        mid16 = mid.astype(jnp.int16)
        msel = jnp.where(key16 <= mid16, jnp.int16(1), jnp.int16(0))
        acc = msel[:, 0:nsub]
        for j in range(1, 16):
            acc = acc + msel[:, j * nsub:(j + 1) * nsub]
        cnt = jnp.sum(acc.astype(jnp.int32), axis=1, keepdims=True)
        ge = cnt >= S
        return jnp.where(ge, lo, mid), jnp.where(ge, mid, hi)

    _, hi = lax.fori_loop(0, 15, it, (lo0, hi0))
    t = lax.bitcast_convert_type((hi + 1) << 16, jnp.float32)
    row = jnp.concatenate([d, jnp.broadcast_to(t, (pb, 16))], axis=1)
    dist_ref[...] = jnp.reshape(row, (pb, 1, n + 16))


def _distthr(xyz_t, new_xyz, rot9, *, pb=16):
    bsz, _, n = xyz_t.shape
    p = new_xyz.shape[1]
    grid = (bsz, p // pb)
    return pl.pallas_call(
        functools.partial(_distthr_body, n=n, pb=pb),
        grid=grid,
        in_specs=[
            pl.BlockSpec((1, 3, n), lambda b, j: (b, 0, 0)),
            pl.BlockSpec((1, pb, 3), lambda b, j: (b, j, 0)),
            pl.BlockSpec((1, pb, 9), lambda b, j: (b, j, 0)),
        ],
        out_specs=pl.BlockSpec((pb, 1, n + 16),
                               lambda b, j: (b * (p // pb) + j, 0, 0)),
        out_shape=jax.ShapeDtypeStruct((bsz * p, 1, n + 16), jnp.float32),
    )(xyz_t, new_xyz, rot9)


def _kminmax(ak, ai, bk, bi):
    sel = ak <= bk
    return (jnp.where(sel, ak, bk), jnp.where(sel, ai, bi),
            jnp.where(sel, bk, ak), jnp.where(sel, bi, ai))


def _bitonic_fix(run):
    n = len(run)
    if n == 1:
        k, i = run[0]
        kk, ii = plsc.sort_key_val(k, i)
        return [(kk, ii)]
    h = n // 2
    lo, hi = [], []
    for j in range(h):
        lk, li, hk, hi_i = _kminmax(run[j][0], run[j][1],
                                    run[j + h][0], run[j + h][1])
        lo.append((lk, li))
        hi.append((hk, hi_i))
    return _bitonic_fix(lo) + _bitonic_fix(hi)


def _merge(a, b, trunc=False):
    n = len(a)
    bp = [(lax.rev(b[n - 1 - j][0], (0,)), lax.rev(b[n - 1 - j][1], (0,)))
          for j in range(n)]
    lo, hi = [], []
    for j in range(n):
        lk, li, hk, hi_i = _kminmax(a[j][0], a[j][1], bp[j][0], bp[j][1])
        lo.append((lk, li))
        hi.append((hk, hi_i))
    if trunc:
        return _bitonic_fix(lo)
    return _bitonic_fix(lo) + _bitonic_fix(hi)


def _sort_lowest64(pairs):
    runs = [[plsc.sort_key_val(k, i)] for k, i in pairs]
    while len(runs) > 1:
        trunc = len(runs[0]) >= 4
        runs = [_merge(runs[2 * j], runs[2 * j + 1], trunc=trunc)
                for j in range(len(runs) // 2)]
    return runs[0]


def _extract(dist, xyz_flat, ftab, *, n, c, bsz):
    rows = dist.shape[0]
    rpw = rows // _NW
    mesh = plsc.VectorSubcoreMesh(core_axis_name="c", subcore_axis_name="s")
    nchunk = n // 16

    def body(dist_hbm, xyzf_hbm, ftab_hbm, fout_hbm, xout_hbm,
             x0_v, x1_v, x2_v, da_v, db_v, svv_v, svi_v, gb_v, fb_v, xo_v,
             semda, semdb, semf, semwf, semwx):
        wid = lax.axis_index("s") * _NC + lax.axis_index("c")
        b = wid // (_NW // bsz)
        base_row = wid * rpw
        pltpu.async_copy(dist_hbm.at[base_row], da_v, semda)
        pltpu.sync_copy(xyzf_hbm.at[pl.ds((b * 3 + 0) * n, n)], x0_v)
        pltpu.sync_copy(xyzf_hbm.at[pl.ds((b * 3 + 1) * n, n)], x1_v)
        pltpu.sync_copy(xyzf_hbm.at[pl.ds((b * 3 + 2) * n, n)], x2_v)
        iota = lax.iota(jnp.int32, 16)
        cutv = jnp.full((16,), VALID_CUT, jnp.float32)
        capv = jnp.full((16,), CAP, jnp.int32)
        last_row = base_row + rpw - 1

        def process(row, drow_v, dsem, nxt_v, nsem):
            @pl.when(row > base_row)
            def _():
                pltpu.make_async_copy(fb_v, fout_hbm.at[pl.ds(0, S)],
                                      semwf).wait()
                pltpu.make_async_copy(xo_v, xout_hbm.at[pl.ds(0, S)],
                                      semwx).wait()

            pltpu.make_async_copy(dist_hbm.at[base_row], drow_v, dsem).wait()
            nr = jnp.minimum(row + 1, last_row)
            pltpu.async_copy(dist_hbm.at[nr], nxt_v, nsem)
            tv = drow_v[0, pl.ds(n, 16)]
            padk = jnp.full((16,), PAD, jnp.float32)
            padi = jnp.zeros((16,), jnp.int32)
            for j in range(CAP // 16 + 1):
                svv_v[pl.ds(j * 16, 16)] = padk
                svi_v[pl.ds(j * 16, 16)] = padi

            def scan(cc, offv):
                base = cc * 32
                off = offv
                for u in range(2):
                    v = drow_v[0, pl.ds(base + u * 16, 16)]
                    m = (v <= tv) & (v < cutv)
                    mi = m.astype(jnp.int32)
                    pos = off + plsc.cumsum(mi) - mi
                    plsc.store_scatter(svv_v, [pos], v, mask=m)
                    plsc.store_scatter(svi_v, [pos], iota + base + u * 16,
                                       mask=m)
                    cnt = plsc.all_reduce_population_count(m)
                    off = jnp.minimum(off + cnt, capv)
                return off

            lax.fori_loop(0, nchunk // 2, scan, jnp.zeros((16,), jnp.int32))

            pairs = [(svv_v[pl.ds(j * 16, 16)], svi_v[pl.ds(j * 16, 16)])
                     for j in range(CAP // 16)]
            low = _sort_lowest64(pairs)
            first = jnp.max(jnp.where(iota == 0, low[0][1], 0))
            fsplat = jnp.full((16,), first, jnp.int32)
            sel = [jnp.where(k < cutv, ii, fsplat) for k, ii in low]
            for j in range(4):
                gb_v[pl.ds(j * 16, 16)] = sel[j] + b * n
            cp = pltpu.async_copy(ftab_hbm.at[gb_v], fb_v, semf)
            zero16 = jnp.zeros((16,), jnp.int32)
            one16 = jnp.full((16,), 1, jnp.int32)
            two16 = jnp.full((16,), 2, jnp.int32)
            for j in range(4):
                rows16 = iota + j * 16
                gx = plsc.load_gather(x0_v, [sel[j]])
                gy = plsc.load_gather(x1_v, [sel[j]])
                gz = plsc.load_gather(x2_v, [sel[j]])
                plsc.store_scatter(xo_v, [rows16, zero16], gx)
                plsc.store_scatter(xo_v, [rows16, one16], gy)
                plsc.store_scatter(xo_v, [rows16, two16], gz)
            cp.wait()
            pltpu.async_copy(fb_v, fout_hbm.at[pl.ds(row * S, S)], semwf)
            pltpu.async_copy(xo_v, xout_hbm.at[pl.ds(row * S, S)], semwx)

        def row_pair(i, _):
            row_a = base_row + 2 * i
            process(row_a, da_v, semda, db_v, semdb)
            process(row_a + 1, db_v, semdb, da_v, semda)
            return 0

        lax.fori_loop(0, rpw // 2, row_pair, 0)
        pltpu.make_async_copy(fb_v, fout_hbm.at[pl.ds(0, S)], semwf).wait()
        pltpu.make_async_copy(xo_v, xout_hbm.at[pl.ds(0, S)], semwx).wait()
        pltpu.make_async_copy(dist_hbm.at[base_row], da_v, semda).wait()

    fn = pl.kernel(
        body,
        out_type=(
            jax.ShapeDtypeStruct((rows * S, c), jnp.float32),
            jax.ShapeDtypeStruct((rows * S, XW), jnp.float32),
        ),
        mesh=mesh,
        compiler_params=pltpu.CompilerParams(needs_layout_passes=False),
        scratch_types=[
            pltpu.VMEM((n,), jnp.float32),
            pltpu.VMEM((n,), jnp.float32),
            pltpu.VMEM((n,), jnp.float32),
            pltpu.VMEM((1, n + 16), jnp.float32),
            pltpu.VMEM((1, n + 16), jnp.float32),
            pltpu.VMEM((CAP + 16,), jnp.float32),
            pltpu.VMEM((CAP + 16,), jnp.int32),
            pltpu.VMEM((S,), jnp.int32),
            pltpu.VMEM((S, c), jnp.float32),
            pltpu.VMEM((S, XW), jnp.float32),
            pltpu.SemaphoreType.DMA,
            pltpu.SemaphoreType.DMA,
            pltpu.SemaphoreType.DMA,
            pltpu.SemaphoreType.DMA,
            pltpu.SemaphoreType.DMA,
        ],
    )
    return fn(dist, xyz_flat, ftab)


def _fixup_body(f_ref, x_ref, c_ref, r_ref, out_ref, *, pg, cch):
    w = pg * S
    feats = jnp.transpose(f_ref[0, 0])
    xt = jnp.transpose(x_ref[0, 0])
    lane = lax.broadcasted_iota(jnp.int32, (1, w), 1)
    pidx = lane // S

    def expand(read):
        acc = jnp.full((1, w), read(0), jnp.float32)
        for p in range(1, pg):
            acc = jnp.where(pidx == p, read(p), acc)
        return acc

    bf = lambda a: a.astype(jnp.bfloat16).astype(jnp.float32)
    rel = [bf(xt[d:d + 1, :] - expand(lambda p, d=d: c_ref[0, p, d]))
           for d in range(3)]
    outs = []
    for d in range(3):
        acc = rel[0] * bf(expand(lambda p, d=d: r_ref[0, p, d]))
        acc = acc + rel[1] * bf(expand(lambda p, d=d: r_ref[0, p, 3 + d]))
        acc = acc + rel[2] * bf(expand(lambda p, d=d: r_ref[0, p, 6 + d]))
        outs.append(acc)
    res = jnp.concatenate(outs + [feats], axis=0)
    out_ref[0] = jnp.reshape(res, (3 + cch, pg, S))


def _fixup(frows, xrows, new_xyz, rot9, *, pg=8):
    bsz, p, _ = new_xyz.shape
    cch = frows.shape[-1]
    fr = frows.reshape(bsz, p // pg, pg * S, cch)
    xr = xrows.reshape(bsz, p // pg, pg * S, xrows.shape[-1])
    grid = (bsz, p // pg)
    out = pl.pallas_call(
        functools.partial(_fixup_body, pg=pg, cch=cch),
        grid=grid,
        in_specs=[
            pl.BlockSpec((1, 1, pg * S, cch), lambda b, j: (b, j, 0, 0)),
            pl.BlockSpec((1, 1, pg * S, xr.shape[-1]), lambda b, j: (b, j, 0, 0)),
            pl.BlockSpec((1, pg, 3), lambda b, j: (b, j, 0),
                         memory_space=pltpu.SMEM),
            pl.BlockSpec((1, pg, 9), lambda b, j: (b, j, 0),
                         memory_space=pltpu.SMEM),
        ],
        out_specs=pl.BlockSpec((1, 3 + cch, pg, S), lambda b, j: (b, 0, j, 0)),
        out_shape=jax.ShapeDtypeStruct((bsz, 3 + cch, p, S), jnp.float32),
    )(fr, xr, new_xyz, rot9)
    return out


def kernel(xyz, new_xyz, rot, features):
    bsz, n, _ = xyz.shape
    p = new_xyz.shape[1]
    c = features.shape[1]

    xyz_t = jnp.transpose(xyz, (0, 2, 1))
    rot9 = rot.reshape(bsz, p, 9)
    dist = _distthr(xyz_t, new_xyz, rot9)

    ftab = jnp.transpose(features, (0, 2, 1)).reshape(bsz * n, c)
    frows, xrows = _extract(dist, xyz_t.reshape(-1), ftab,
                            n=n, c=c, bsz=bsz)

    return _fixup(frows, xrows, new_xyz, rot9)


# speedup vs baseline: 8.3900x; 1.0165x over previous
import functools

import numpy as np
import jax
import jax.numpy as jnp
from jax import lax
from jax.experimental import pallas as pl
from jax.experimental.pallas import tpu as pltpu
from jax.experimental.pallas import tpu_sc as plsc

RADIUS2 = 4.0
HMIN = -1.0
HMAX = 1.0
S = 64
INVALID = 1e10
VALID_CUT = 1e9
PAD = 1e30
CAP = 128
HI_K = int(np.float32(INVALID).view(np.int32)) // 65536 + 1
XW = 8

_NC = 2
_NS = 16
_NW = _NC * _NS


def _distthr_body(xyz_ref, c_ref, r_ref, dist_ref, *, n, pb):
    x = xyz_ref[0, 0:1, :]
    y = xyz_ref[0, 1:2, :]
    z = xyz_ref[0, 2:3, :]
    cx = c_ref[0, :, 0:1]
    cy = c_ref[0, :, 1:2]
    cz = c_ref[0, :, 2:3]
    r = r_ref[0]
    bf = lambda a: a.astype(jnp.bfloat16).astype(jnp.float32)
    dx = bf(x - cx)
    dy = bf(y - cy)
    dz = bf(z - cz)
    rb = bf(r)
    lx = dx * rb[:, 0:1] + dy * rb[:, 3:4] + dz * rb[:, 6:7]
    ly = dx * rb[:, 1:2] + dy * rb[:, 4:5] + dz * rb[:, 7:8]
    lz = dx * rb[:, 2:3] + dy * rb[:, 5:6] + dz * rb[:, 8:9]
    r2 = ly * ly + lz * lz
    in_cyl = (lx >= HMIN) & (lx <= HMAX) & (r2 < RADIUS2)
    d = jnp.where(in_cyl, r2, INVALID)

    key16 = (lax.bitcast_convert_type(d, jnp.int32) >> 16).astype(jnp.int16)
    lo0 = jnp.full((pb, 1), -1, jnp.int32)
    hi0 = jnp.full((pb, 1), HI_K - 1, jnp.int32)

    nsub = n // 16

    def it(_, carry):
        lo, hi = carry
        mid = (lo + hi) >> 1
        mid16 = mid.astype(jnp.int16)
        msel = (key16 <= mid16).astype(jnp.int16)
        acc = msel[:, 0:nsub]
        for j in range(1, 16):
            acc = acc + msel[:, j * nsub:(j + 1) * nsub]
        cnt = jnp.sum(acc.astype(jnp.int32), axis=1, keepdims=True)
        ge = cnt >= S
        return jnp.where(ge, lo, mid), jnp.where(ge, mid, hi)

    _, hi = lax.fori_loop(0, 15, it, (lo0, hi0))
    t = lax.bitcast_convert_type((hi + 1) << 16, jnp.float32)
    row = jnp.concatenate([d, jnp.broadcast_to(t, (pb, 16))], axis=1)
    dist_ref[...] = jnp.reshape(row, (pb, 1, n + 16))


def _distthr(xyz_t, new_xyz, rot9, *, pb=16):
    bsz, _, n = xyz_t.shape
    p = new_xyz.shape[1]
    grid = (bsz, p // pb)
    return pl.pallas_call(
        functools.partial(_distthr_body, n=n, pb=pb),
        grid=grid,
        in_specs=[
            pl.BlockSpec((1, 3, n), lambda b, j: (b, 0, 0)),
            pl.BlockSpec((1, pb, 3), lambda b, j: (b, j, 0)),
            pl.BlockSpec((1, pb, 9), lambda b, j: (b, j, 0)),
        ],
        out_specs=pl.BlockSpec((pb, 1, n + 16),
                               lambda b, j: (b * (p // pb) + j, 0, 0)),
        out_shape=jax.ShapeDtypeStruct((bsz * p, 1, n + 16), jnp.float32),
    )(xyz_t, new_xyz, rot9)


def _kminmax(ak, ai, bk, bi):
    sel = ak <= bk
    return (jnp.where(sel, ak, bk), jnp.where(sel, ai, bi),
            jnp.where(sel, bk, ak), jnp.where(sel, bi, ai))


def _bitonic_fix(run):
    n = len(run)
    if n == 1:
        k, i = run[0]
        kk, ii = plsc.sort_key_val(k, i)
        return [(kk, ii)]
    h = n // 2
    lo, hi = [], []
    for j in range(h):
        lk, li, hk, hi_i = _kminmax(run[j][0], run[j][1],
                                    run[j + h][0], run[j + h][1])
        lo.append((lk, li))
        hi.append((hk, hi_i))
    return _bitonic_fix(lo) + _bitonic_fix(hi)


def _merge(a, b, trunc=False):
    n = len(a)
    bp = [(lax.rev(b[n - 1 - j][0], (0,)), lax.rev(b[n - 1 - j][1], (0,)))
          for j in range(n)]
    lo, hi = [], []
    for j in range(n):
        lk, li, hk, hi_i = _kminmax(a[j][0], a[j][1], bp[j][0], bp[j][1])
        lo.append((lk, li))
        hi.append((hk, hi_i))
    if trunc:
        return _bitonic_fix(lo)
    return _bitonic_fix(lo) + _bitonic_fix(hi)


def _sort_lowest64(pairs):
    runs = [[plsc.sort_key_val(k, i)] for k, i in pairs]
    while len(runs) > 1:
        trunc = len(runs[0]) >= 4
        runs = [_merge(runs[2 * j], runs[2 * j + 1], trunc=trunc)
                for j in range(len(runs) // 2)]
    return runs[0]


def _extract(dist, xyz_flat, ftab, *, n, c, bsz):
    rows = dist.shape[0]
    rpw = rows // _NW
    mesh = plsc.VectorSubcoreMesh(core_axis_name="c", subcore_axis_name="s")
    nchunk = n // 16

    def body(dist_hbm, xyzf_hbm, ftab_hbm, fout_hbm, xout_hbm,
             x0_v, x1_v, x2_v, da_v, db_v, svv_v, svi_v, gb_v, fb_v, xo_v,
             semda, semdb, semf, semwf, semwx):
        wid = lax.axis_index("s") * _NC + lax.axis_index("c")
        b = wid // (_NW // bsz)
        base_row = wid * rpw
        pltpu.async_copy(dist_hbm.at[base_row], da_v, semda)
        pltpu.sync_copy(xyzf_hbm.at[pl.ds((b * 3 + 0) * n, n)], x0_v)
        pltpu.sync_copy(xyzf_hbm.at[pl.ds((b * 3 + 1) * n, n)], x1_v)
        pltpu.sync_copy(xyzf_hbm.at[pl.ds((b * 3 + 2) * n, n)], x2_v)
        iota = lax.iota(jnp.int32, 16)
        cutv = jnp.full((16,), VALID_CUT, jnp.float32)
        capv = jnp.full((16,), CAP, jnp.int32)
        last_row = base_row + rpw - 1

        def process(row, drow_v, dsem, nxt_v, nsem):
            @pl.when(row > base_row)
            def _():
                pltpu.make_async_copy(fb_v, fout_hbm.at[pl.ds(0, S)],
                                      semwf).wait()
                pltpu.make_async_copy(xo_v, xout_hbm.at[pl.ds(0, S)],
                                      semwx).wait()

            pltpu.make_async_copy(dist_hbm.at[base_row], drow_v, dsem).wait()
            nr = jnp.minimum(row + 1, last_row)
            pltpu.async_copy(dist_hbm.at[nr], nxt_v, nsem)
            tv = drow_v[0, pl.ds(n, 16)]
            padk = jnp.full((16,), PAD, jnp.float32)
            padi = jnp.zeros((16,), jnp.int32)
            for j in range(CAP // 16 + 1):
                svv_v[pl.ds(j * 16, 16)] = padk
                svi_v[pl.ds(j * 16, 16)] = padi

            def scan(cc, offv):
                base = cc * 32
                off = offv
                for u in range(2):
                    v = drow_v[0, pl.ds(base + u * 16, 16)]
                    m = (v <= tv) & (v < cutv)
                    mi = m.astype(jnp.int32)
                    pos = off + plsc.cumsum(mi) - mi
                    plsc.store_scatter(svv_v, [pos], v, mask=m)
                    plsc.store_scatter(svi_v, [pos], iota + base + u * 16,
                                       mask=m)
                    cnt = plsc.all_reduce_population_count(m)
                    off = jnp.minimum(off + cnt, capv)
                return off

            lax.fori_loop(0, nchunk // 2, scan, jnp.zeros((16,), jnp.int32))

            pairs = [(svv_v[pl.ds(j * 16, 16)], svi_v[pl.ds(j * 16, 16)])
                     for j in range(CAP // 16)]
            low = _sort_lowest64(pairs)
            first = jnp.max(jnp.where(iota == 0, low[0][1], 0))
            fsplat = jnp.full((16,), first, jnp.int32)
            sel = [jnp.where(k < cutv, ii, fsplat) for k, ii in low]
            for j in range(4):
                gb_v[pl.ds(j * 16, 16)] = sel[j] + b * n
            cp = pltpu.async_copy(ftab_hbm.at[gb_v], fb_v, semf)
            zero16 = jnp.zeros((16,), jnp.int32)
            one16 = jnp.full((16,), 1, jnp.int32)
            two16 = jnp.full((16,), 2, jnp.int32)
            for j in range(4):
                rows16 = iota + j * 16
                gx = plsc.load_gather(x0_v, [sel[j]])
                gy = plsc.load_gather(x1_v, [sel[j]])
                gz = plsc.load_gather(x2_v, [sel[j]])
                plsc.store_scatter(xo_v, [rows16, zero16], gx)
                plsc.store_scatter(xo_v, [rows16, one16], gy)
                plsc.store_scatter(xo_v, [rows16, two16], gz)
            cp.wait()
            pltpu.async_copy(fb_v, fout_hbm.at[pl.ds(row * S, S)], semwf)
            pltpu.async_copy(xo_v, xout_hbm.at[pl.ds(row * S, S)], semwx)

        def row_pair(i, _):
            row_a = base_row + 2 * i
            process(row_a, da_v, semda, db_v, semdb)
            process(row_a + 1, db_v, semdb, da_v, semda)
            return 0

        lax.fori_loop(0, rpw // 2, row_pair, 0)
        pltpu.make_async_copy(fb_v, fout_hbm.at[pl.ds(0, S)], semwf).wait()
        pltpu.make_async_copy(xo_v, xout_hbm.at[pl.ds(0, S)], semwx).wait()
        pltpu.make_async_copy(dist_hbm.at[base_row], da_v, semda).wait()

    fn = pl.kernel(
        body,
        out_type=(
            jax.ShapeDtypeStruct((rows * S, c), jnp.float32),
            jax.ShapeDtypeStruct((rows * S, XW), jnp.float32),
        ),
        mesh=mesh,
        compiler_params=pltpu.CompilerParams(needs_layout_passes=False),
        scratch_types=[
            pltpu.VMEM((n,), jnp.float32),
            pltpu.VMEM((n,), jnp.float32),
            pltpu.VMEM((n,), jnp.float32),
            pltpu.VMEM((1, n + 16), jnp.float32),
            pltpu.VMEM((1, n + 16), jnp.float32),
            pltpu.VMEM((CAP + 16,), jnp.float32),
            pltpu.VMEM((CAP + 16,), jnp.int32),
            pltpu.VMEM((S,), jnp.int32),
            pltpu.VMEM((S, c), jnp.float32),
            pltpu.VMEM((S, XW), jnp.float32),
            pltpu.SemaphoreType.DMA,
            pltpu.SemaphoreType.DMA,
            pltpu.SemaphoreType.DMA,
            pltpu.SemaphoreType.DMA,
            pltpu.SemaphoreType.DMA,
        ],
    )
    return fn(dist, xyz_flat, ftab)


def _fixup_body(f_ref, x_ref, c_ref, r_ref, out_ref, *, pg, cch):
    w = pg * S
    feats = jnp.transpose(f_ref[0, 0])
    xt = jnp.transpose(x_ref[0, 0])
    lane = lax.broadcasted_iota(jnp.int32, (1, w), 1)
    pidx = lane // S

    def expand(read):
        acc = jnp.full((1, w), read(0), jnp.float32)
        for p in range(1, pg):
            acc = jnp.where(pidx == p, read(p), acc)
        return acc

    bf = lambda a: a.astype(jnp.bfloat16).astype(jnp.float32)
    rel = [bf(xt[d:d + 1, :] - expand(lambda p, d=d: c_ref[0, p, d]))
           for d in range(3)]
    outs = []
    for d in range(3):
        acc = rel[0] * bf(expand(lambda p, d=d: r_ref[0, p, d]))
        acc = acc + rel[1] * bf(expand(lambda p, d=d: r_ref[0, p, 3 + d]))
        acc = acc + rel[2] * bf(expand(lambda p, d=d: r_ref[0, p, 6 + d]))
        outs.append(acc)
    res = jnp.concatenate(outs + [feats], axis=0)
    out_ref[0] = jnp.reshape(res, (3 + cch, pg, S))


def _fixup(frows, xrows, new_xyz, rot9, *, pg=8):
    bsz, p, _ = new_xyz.shape
    cch = frows.shape[-1]
    fr = frows.reshape(bsz, p // pg, pg * S, cch)
    xr = xrows.reshape(bsz, p // pg, pg * S, xrows.shape[-1])
    grid = (bsz, p // pg)
    out = pl.pallas_call(
        functools.partial(_fixup_body, pg=pg, cch=cch),
        grid=grid,
        in_specs=[
            pl.BlockSpec((1, 1, pg * S, cch), lambda b, j: (b, j, 0, 0)),
            pl.BlockSpec((1, 1, pg * S, xr.shape[-1]), lambda b, j: (b, j, 0, 0)),
            pl.BlockSpec((1, pg, 3), lambda b, j: (b, j, 0),
                         memory_space=pltpu.SMEM),
            pl.BlockSpec((1, pg, 9), lambda b, j: (b, j, 0),
                         memory_space=pltpu.SMEM),
        ],
        out_specs=pl.BlockSpec((1, 3 + cch, pg, S), lambda b, j: (b, 0, j, 0)),
        out_shape=jax.ShapeDtypeStruct((bsz, 3 + cch, p, S), jnp.float32),
    )(fr, xr, new_xyz, rot9)
    return out


def kernel(xyz, new_xyz, rot, features):
    bsz, n, _ = xyz.shape
    p = new_xyz.shape[1]
    c = features.shape[1]

    xyz_t = jnp.transpose(xyz, (0, 2, 1))
    rot9 = rot.reshape(bsz, p, 9)

    frs, xrs = [], []
    for bi in range(bsz):
        xt_b = xyz_t[bi:bi + 1]
        d_b = _distthr(xt_b, new_xyz[bi:bi + 1], rot9[bi:bi + 1])
        ftab_b = jnp.transpose(features[bi], (1, 0))
        fr, xr = _extract(d_b, xt_b.reshape(-1), ftab_b, n=n, c=c, bsz=1)
        frs.append(fr)
        xrs.append(xr)
    frows = jnp.concatenate(frs, axis=0)
    xrows = jnp.concatenate(xrs, axis=0)

    return _fixup(frows, xrows, new_xyz, rot9)


# speedup vs baseline: 9.0200x; 1.0751x over previous
import functools

import numpy as np
import jax
import jax.numpy as jnp
from jax import lax
from jax.experimental import pallas as pl
from jax.experimental.pallas import tpu as pltpu
from jax.experimental.pallas import tpu_sc as plsc

RADIUS2 = 4.0
HMIN = -1.0
HMAX = 1.0
S = 64
INVALID = 1e10
VALID_CUT = 1e9
PAD = 1e30
CAP = 128
HI_K = int(np.float32(INVALID).view(np.int32)) // 65536 + 1
XW = 8
NSLOT = 128
SVN = 336

_NC = 2
_NS = 16
_NW = _NC * _NS


def _distthr_body(xyz_ref, c_ref, r_ref, dist_ref, cmeta_ref, *, n, pb):
    x = xyz_ref[0, 0:1, :]
    y = xyz_ref[0, 1:2, :]
    z = xyz_ref[0, 2:3, :]
    cx = c_ref[0, :, 0:1]
    cy = c_ref[0, :, 1:2]
    cz = c_ref[0, :, 2:3]
    r = r_ref[0]
    bf = lambda a: a.astype(jnp.bfloat16).astype(jnp.float32)
    dx = bf(x - cx)
    dy = bf(y - cy)
    dz = bf(z - cz)
    rb = bf(r)
    lx = dx * rb[:, 0:1] + dy * rb[:, 3:4] + dz * rb[:, 6:7]
    ly = dx * rb[:, 1:2] + dy * rb[:, 4:5] + dz * rb[:, 7:8]
    lz = dx * rb[:, 2:3] + dy * rb[:, 5:6] + dz * rb[:, 8:9]
    r2 = ly * ly + lz * lz
    in_cyl = (lx >= HMIN) & (lx <= HMAX) & (r2 < RADIUS2)
    d = jnp.where(in_cyl, r2, INVALID)

    key16 = (lax.bitcast_convert_type(d, jnp.int32) >> 16).astype(jnp.int16)
    lo0 = jnp.full((pb, 1), -1, jnp.int32)
    hi0 = jnp.full((pb, 1), HI_K - 1, jnp.int32)

    nsub = n // 16

    def it(_, carry):
        lo, hi = carry
        mid = (lo + hi) >> 1
        mid16 = mid.astype(jnp.int16)
        msel = (key16 <= mid16).astype(jnp.int16)
        acc = msel[:, 0:nsub]
        for j in range(1, 16):
            acc = acc + msel[:, j * nsub:(j + 1) * nsub]
        cnt = jnp.sum(acc.astype(jnp.int32), axis=1, keepdims=True)
        ge = cnt >= S
        return jnp.where(ge, lo, mid), jnp.where(ge, mid, hi)

    _, hi = lax.fori_loop(0, 15, it, (lo0, hi0))
    t = lax.bitcast_convert_type((hi + 1) << 16, jnp.float32)
    row = jnp.concatenate([d, jnp.broadcast_to(t, (pb, 16))], axis=1)
    dist_ref[...] = jnp.reshape(row, (pb, 1, n + 16))

    nch64 = n // 64
    m = (d <= t) & (d < VALID_CUT)
    c64 = jnp.sum(m.astype(jnp.float32).reshape(pb, nch64, 64), axis=2)
    hot = (c64 > 0).astype(jnp.float32)

    ii = lax.broadcasted_iota(jnp.int32, (nch64, nch64), 0)
    jj = lax.broadcasted_iota(jnp.int32, (nch64, nch64), 1)
    tri = jnp.where(ii < jj, 1.0, 0.0)
    ex = jnp.dot(hot, tri)
    cbase = jnp.minimum(jnp.dot(c64, tri), 256.0)
    hotcnt = jnp.sum(hot, axis=1, keepdims=True)
    sl = lax.broadcasted_iota(jnp.int32, (1, 1, NSLOT), 2).astype(jnp.float32)
    oh = jnp.where((ex[:, :, None] == sl) & (hot[:, :, None] > 0), 1.0, 0.0)
    jidx = lax.broadcasted_iota(jnp.int32, (pb, nch64, 1), 1).astype(jnp.float32)
    cid = jnp.sum(oh * jidx, axis=1)
    base = jnp.sum(oh * cbase[:, :, None], axis=1)
    meta = (cid * 512.0 + base).astype(jnp.int32)
    lane = lax.broadcasted_iota(jnp.int32, (pb, NSLOT), 1)
    cnt_i = jnp.minimum(hotcnt, float(NSLOT - 1)).astype(jnp.int32)
    meta = jnp.where(lane == NSLOT - 1, cnt_i, meta)
    cmeta_ref[...] = jnp.reshape(meta, (pb, 1, NSLOT))


def _distthr(xyz_t, new_xyz, rot9, *, pb=16):
    bsz, _, n = xyz_t.shape
    p = new_xyz.shape[1]
    grid = (bsz, p // pb)
    return pl.pallas_call(
        functools.partial(_distthr_body, n=n, pb=pb),
        grid=grid,
        in_specs=[
            pl.BlockSpec((1, 3, n), lambda b, j: (b, 0, 0)),
            pl.BlockSpec((1, pb, 3), lambda b, j: (b, j, 0)),
            pl.BlockSpec((1, pb, 9), lambda b, j: (b, j, 0)),
        ],
        out_specs=[
            pl.BlockSpec((pb, 1, n + 16),
                         lambda b, j: (b * (p // pb) + j, 0, 0)),
            pl.BlockSpec((pb, 1, NSLOT),
                         lambda b, j: (b * (p // pb) + j, 0, 0)),
        ],
        out_shape=[
            jax.ShapeDtypeStruct((bsz * p, 1, n + 16), jnp.float32),
            jax.ShapeDtypeStruct((bsz * p, 1, NSLOT), jnp.int32),
        ],
    )(xyz_t, new_xyz, rot9)


def _kminmax(ak, ai, bk, bi):
    sel = ak <= bk
    return (jnp.where(sel, ak, bk), jnp.where(sel, ai, bi),
            jnp.where(sel, bk, ak), jnp.where(sel, bi, ai))


def _bitonic_fix(run):
    n = len(run)
    if n == 1:
        k, i = run[0]
        kk, ii = plsc.sort_key_val(k, i)
        return [(kk, ii)]
    h = n // 2
    lo, hi = [], []
    for j in range(h):
        lk, li, hk, hi_i = _kminmax(run[j][0], run[j][1],
                                    run[j + h][0], run[j + h][1])
        lo.append((lk, li))
        hi.append((hk, hi_i))
    return _bitonic_fix(lo) + _bitonic_fix(hi)


def _merge(a, b, trunc=False):
    n = len(a)
    bp = [(lax.rev(b[n - 1 - j][0], (0,)), lax.rev(b[n - 1 - j][1], (0,)))
          for j in range(n)]
    lo, hi = [], []
    for j in range(n):
        lk, li, hk, hi_i = _kminmax(a[j][0], a[j][1], bp[j][0], bp[j][1])
        lo.append((lk, li))
        hi.append((hk, hi_i))
    if trunc:
        return _bitonic_fix(lo)
    return _bitonic_fix(lo) + _bitonic_fix(hi)


def _sort_lowest64(pairs):
    runs = [[plsc.sort_key_val(k, i)] for k, i in pairs]
    while len(runs) > 1:
        trunc = len(runs[0]) >= 4
        runs = [_merge(runs[2 * j], runs[2 * j + 1], trunc=trunc)
                for j in range(len(runs) // 2)]
    return runs[0]


def _extract(dist, cmeta, xyz_flat, ftab, *, n, c, bsz):
    rows = dist.shape[0]
    rpw = rows // _NW
    mesh = plsc.VectorSubcoreMesh(core_axis_name="c", subcore_axis_name="s")

    def body(dist_hbm, cmeta_hbm, xyzf_hbm, ftab_hbm, fout_hbm, xout_hbm,
             x0_v, x1_v, x2_v, da_v, db_v, svv_v, svi_v, gb_v, fb_v, xo_v,
             cm_s,
             semda, semdb, semf, semwf, semwx):
        wid = lax.axis_index("s") * _NC + lax.axis_index("c")
        b = wid // (_NW // bsz)
        base_row = wid * rpw
        pltpu.async_copy(dist_hbm.at[base_row], da_v, semda)
        pltpu.sync_copy(xyzf_hbm.at[pl.ds((b * 3 + 0) * n, n)], x0_v)
        pltpu.sync_copy(xyzf_hbm.at[pl.ds((b * 3 + 1) * n, n)], x1_v)
        pltpu.sync_copy(xyzf_hbm.at[pl.ds((b * 3 + 2) * n, n)], x2_v)
        iota = lax.iota(jnp.int32, 16)
        cutv = jnp.full((16,), VALID_CUT, jnp.float32)
        capv = jnp.full((16,), CAP, jnp.int32)
        last_row = base_row + rpw - 1

        def process(row, drow_v, dsem, nxt_v, nsem):
            @pl.when(row > base_row)
            def _():
                pltpu.make_async_copy(fb_v, fout_hbm.at[pl.ds(0, S)],
                                      semwf).wait()
                pltpu.make_async_copy(xo_v, xout_hbm.at[pl.ds(0, S)],
                                      semwx).wait()

            pltpu.make_async_copy(dist_hbm.at[base_row], drow_v, dsem).wait()
            nr = jnp.minimum(row + 1, last_row)
            pltpu.async_copy(dist_hbm.at[nr], nxt_v, nsem)
            pltpu.sync_copy(cmeta_hbm.at[row],
                            cm_s.at[pl.ds(0, 1), pl.ds(0, NSLOT)])
            tv = drow_v[0, pl.ds(n, 16)]
            padk = jnp.full((16,), PAD, jnp.float32)
            padi = jnp.zeros((16,), jnp.int32)
            for j in range(CAP // 16 + 1):
                svv_v[pl.ds(j * 16, 16)] = padk
                svi_v[pl.ds(j * 16, 16)] = padi

            hotcnt = cm_s[0, pl.ds(NSLOT - 16, 16)][15]

            def slot(s, _):
                meta = cm_s[0, pl.ds(s, 16)][0]
                ch = meta >> 9
                cb = meta & 511
                off = jnp.full((16,), cb, jnp.int32)
                for u in range(4):
                    lo = ch * 64 + u * 16
                    v = drow_v[0, pl.ds(lo, 16)]
                    m = (v <= tv) & (v < cutv)
                    mi = m.astype(jnp.int32)
                    pos = off + plsc.cumsum(mi) - mi
                    plsc.store_scatter(svv_v, [pos], v, mask=m)
                    plsc.store_scatter(svi_v, [pos], iota + lo, mask=m)
                    cnt = plsc.all_reduce_population_count(m)
                    off = off + cnt
                return 0

            lax.fori_loop(0, hotcnt, slot, 0)

            pairs = [(svv_v[pl.ds(j * 16, 16)], svi_v[pl.ds(j * 16, 16)])
                     for j in range(CAP // 16)]
            low = _sort_lowest64(pairs)
            first = jnp.max(jnp.where(iota == 0, low[0][1], 0))
            fsplat = jnp.full((16,), first, jnp.int32)
            sel = [jnp.where(k < cutv, ii, fsplat) for k, ii in low]
            for j in range(4):
                gb_v[pl.ds(j * 16, 16)] = sel[j] + b * n
            cp = pltpu.async_copy(ftab_hbm.at[gb_v], fb_v, semf)
            zero16 = jnp.zeros((16,), jnp.int32)
            one16 = jnp.full((16,), 1, jnp.int32)
            two16 = jnp.full((16,), 2, jnp.int32)
            for j in range(4):
                rows16 = iota + j * 16
                gx = plsc.load_gather(x0_v, [sel[j]])
                gy = plsc.load_gather(x1_v, [sel[j]])
                gz = plsc.load_gather(x2_v, [sel[j]])
                plsc.store_scatter(xo_v, [rows16, zero16], gx)
                plsc.store_scatter(xo_v, [rows16, one16], gy)
                plsc.store_scatter(xo_v, [rows16, two16], gz)
            cp.wait()
            pltpu.async_copy(fb_v, fout_hbm.at[pl.ds(row * S, S)], semwf)
            pltpu.async_copy(xo_v, xout_hbm.at[pl.ds(row * S, S)], semwx)

        def row_pair(i, _):
            row_a = base_row + 2 * i
            process(row_a, da_v, semda, db_v, semdb)
            process(row_a + 1, db_v, semdb, da_v, semda)
            return 0

        lax.fori_loop(0, rpw // 2, row_pair, 0)
        pltpu.make_async_copy(fb_v, fout_hbm.at[pl.ds(0, S)], semwf).wait()
        pltpu.make_async_copy(xo_v, xout_hbm.at[pl.ds(0, S)], semwx).wait()
        pltpu.make_async_copy(dist_hbm.at[base_row], da_v, semda).wait()

    fn = pl.kernel(
        body,
        out_type=(
            jax.ShapeDtypeStruct((rows * S, c), jnp.float32),
            jax.ShapeDtypeStruct((rows * S, XW), jnp.float32),
        ),
        mesh=mesh,
        compiler_params=pltpu.CompilerParams(needs_layout_passes=False),
        scratch_types=[
            pltpu.VMEM((n,), jnp.float32),
            pltpu.VMEM((n,), jnp.float32),
            pltpu.VMEM((n,), jnp.float32),
            pltpu.VMEM((1, n + 16), jnp.float32),
            pltpu.VMEM((1, n + 16), jnp.float32),
            pltpu.VMEM((SVN,), jnp.float32),
            pltpu.VMEM((SVN,), jnp.int32),
            pltpu.VMEM((S,), jnp.int32),
            pltpu.VMEM((S, c), jnp.float32),
            pltpu.VMEM((S, XW), jnp.float32),
            pltpu.VMEM((1, NSLOT + 16), jnp.int32),
            pltpu.SemaphoreType.DMA,
            pltpu.SemaphoreType.DMA,
            pltpu.SemaphoreType.DMA,
            pltpu.SemaphoreType.DMA,
            pltpu.SemaphoreType.DMA,
        ],
    )
    return fn(dist, cmeta, xyz_flat, ftab)


def _fixup_body(f_ref, x_ref, c_ref, r_ref, out_ref, *, pg, cch):
    w = pg * S
    feats = jnp.transpose(f_ref[0, 0])
    xt = jnp.transpose(x_ref[0, 0])
    lane = lax.broadcasted_iota(jnp.int32, (1, w), 1)
    pidx = lane // S

    def expand(read):
        acc = jnp.full((1, w), read(0), jnp.float32)
        for p in range(1, pg):
            acc = jnp.where(pidx == p, read(p), acc)
        return acc

    bf = lambda a: a.astype(jnp.bfloat16).astype(jnp.float32)
    rel = [bf(xt[d:d + 1, :] - expand(lambda p, d=d: c_ref[0, p, d]))
           for d in range(3)]
    outs = []
    for d in range(3):
        acc = rel[0] * bf(expand(lambda p, d=d: r_ref[0, p, d]))
        acc = acc + rel[1] * bf(expand(lambda p, d=d: r_ref[0, p, 3 + d]))
        acc = acc + rel[2] * bf(expand(lambda p, d=d: r_ref[0, p, 6 + d]))
        outs.append(acc)
    res = jnp.concatenate(outs + [feats], axis=0)
    out_ref[0] = jnp.reshape(res, (3 + cch, pg, S))


def _fixup(frows, xrows, new_xyz, rot9, *, pg=8):
    bsz, p, _ = new_xyz.shape
    cch = frows.shape[-1]
    fr = frows.reshape(bsz, p // pg, pg * S, cch)
    xr = xrows.reshape(bsz, p // pg, pg * S, xrows.shape[-1])
    grid = (bsz, p // pg)
    out = pl.pallas_call(
        functools.partial(_fixup_body, pg=pg, cch=cch),
        grid=grid,
        in_specs=[
            pl.BlockSpec((1, 1, pg * S, cch), lambda b, j: (b, j, 0, 0)),
            pl.BlockSpec((1, 1, pg * S, xr.shape[-1]), lambda b, j: (b, j, 0, 0)),
            pl.BlockSpec((1, pg, 3), lambda b, j: (b, j, 0),
                         memory_space=pltpu.SMEM),
            pl.BlockSpec((1, pg, 9), lambda b, j: (b, j, 0),
                         memory_space=pltpu.SMEM),
        ],
        out_specs=pl.BlockSpec((1, 3 + cch, pg, S), lambda b, j: (b, 0, j, 0)),
        out_shape=jax.ShapeDtypeStruct((bsz, 3 + cch, p, S), jnp.float32),
    )(fr, xr, new_xyz, rot9)
    return out


def kernel(xyz, new_xyz, rot, features):
    bsz, n, _ = xyz.shape
    p = new_xyz.shape[1]
    c = features.shape[1]

    xyz_t = jnp.transpose(xyz, (0, 2, 1))
    rot9 = rot.reshape(bsz, p, 9)

    frs, xrs = [], []
    for bi in range(bsz):
        xt_b = xyz_t[bi:bi + 1]
        d_b, cm_b = _distthr(xt_b, new_xyz[bi:bi + 1], rot9[bi:bi + 1])
        ftab_b = jnp.transpose(features[bi], (1, 0))
        fr, xr = _extract(d_b, cm_b, xt_b.reshape(-1), ftab_b,
                          n=n, c=c, bsz=1)
        frs.append(fr)
        xrs.append(xr)
    frows = jnp.concatenate(frs, axis=0)
    xrows = jnp.concatenate(xrs, axis=0)

    return _fixup(frows, xrows, new_xyz, rot9)


# speedup vs baseline: 9.1958x; 1.0195x over previous
import functools

import numpy as np
import jax
import jax.numpy as jnp
from jax import lax
from jax.experimental import pallas as pl
from jax.experimental.pallas import tpu as pltpu
from jax.experimental.pallas import tpu_sc as plsc

RADIUS2 = 4.0
HMIN = -1.0
HMAX = 1.0
S = 64
INVALID = 1e10
VALID_CUT = 1e9
PAD = 1e30
CAP = 128
HI_K = int(np.float32(INVALID).view(np.int32)) // 65536 + 1
XW = 8
NSLOT = 128
SVN = 336

_NC = 2
_NS = 16
_NW = _NC * _NS


def _distthr_body(xyz_ref, c_ref, r_ref, dist_ref, cmeta_ref, *, n, pb):
    x = xyz_ref[0, 0:1, :]
    y = xyz_ref[0, 1:2, :]
    z = xyz_ref[0, 2:3, :]
    cx = c_ref[0, :, 0:1]
    cy = c_ref[0, :, 1:2]
    cz = c_ref[0, :, 2:3]
    r = r_ref[0]
    bf = lambda a: a.astype(jnp.bfloat16).astype(jnp.float32)
    dx = bf(x - cx)
    dy = bf(y - cy)
    dz = bf(z - cz)
    rb = bf(r)
    lx = dx * rb[:, 0:1] + dy * rb[:, 3:4] + dz * rb[:, 6:7]
    ly = dx * rb[:, 1:2] + dy * rb[:, 4:5] + dz * rb[:, 7:8]
    lz = dx * rb[:, 2:3] + dy * rb[:, 5:6] + dz * rb[:, 8:9]
    r2 = ly * ly + lz * lz
    in_cyl = (lx >= HMIN) & (lx <= HMAX) & (r2 < RADIUS2)
    d = jnp.where(in_cyl, r2, INVALID)

    key16 = (lax.bitcast_convert_type(d, jnp.int32) >> 16).astype(jnp.int16)
    lo0 = jnp.full((pb, 1), -1, jnp.int32)
    hi0 = jnp.full((pb, 1), HI_K - 1, jnp.int32)

    nsub = n // 16

    def it(_, carry):
        lo, hi = carry
        mid = (lo + hi) >> 1
        mid16 = mid.astype(jnp.int16)
        msel = (key16 <= mid16).astype(jnp.int16)
        acc = msel[:, 0:nsub]
        for j in range(1, 16):
            acc = acc + msel[:, j * nsub:(j + 1) * nsub]
        cnt = jnp.sum(acc.astype(jnp.int32), axis=1, keepdims=True)
        ge = cnt >= S
        return jnp.where(ge, lo, mid), jnp.where(ge, mid, hi)

    _, hi = lax.fori_loop(0, 13, it, (lo0, hi0))
    t = lax.bitcast_convert_type((hi + 1) << 16, jnp.float32)
    row = jnp.concatenate([d, jnp.broadcast_to(t, (pb, 16))], axis=1)
    dist_ref[...] = jnp.reshape(row, (pb, 1, n + 16))

    nch64 = n // 64
    m = (d <= t) & (d < VALID_CUT)
    c64 = jnp.sum(m.astype(jnp.float32).reshape(pb, nch64, 64), axis=2)
    hot = (c64 > 0).astype(jnp.float32)

    ii = lax.broadcasted_iota(jnp.int32, (nch64, nch64), 0)
    jj = lax.broadcasted_iota(jnp.int32, (nch64, nch64), 1)
    tri = jnp.where(ii < jj, 1.0, 0.0)
    ex = jnp.dot(hot, tri)
    cbase = jnp.minimum(jnp.dot(c64, tri), 256.0)
    hotcnt = jnp.sum(hot, axis=1, keepdims=True)
    sl = lax.broadcasted_iota(jnp.int32, (1, 1, NSLOT), 2).astype(jnp.float32)
    oh = jnp.where((ex[:, :, None] == sl) & (hot[:, :, None] > 0), 1.0, 0.0)
    jidx = lax.broadcasted_iota(jnp.int32, (pb, nch64, 1), 1).astype(jnp.float32)
    cid = jnp.sum(oh * jidx, axis=1)
    base = jnp.sum(oh * cbase[:, :, None], axis=1)
    meta = (cid * 512.0 + base).astype(jnp.int32)
    lane = lax.broadcasted_iota(jnp.int32, (pb, NSLOT), 1)
    cnt_i = jnp.minimum(hotcnt, float(NSLOT - 1)).astype(jnp.int32)
    meta = jnp.where(lane == NSLOT - 1, cnt_i, meta)
    cmeta_ref[...] = jnp.reshape(meta, (pb, 1, NSLOT))


def _distthr(xyz_t, new_xyz, rot9, *, pb=16):
    bsz, _, n = xyz_t.shape
    p = new_xyz.shape[1]
    grid = (bsz, p // pb)
    return pl.pallas_call(
        functools.partial(_distthr_body, n=n, pb=pb),
        grid=grid,
        in_specs=[
            pl.BlockSpec((1, 3, n), lambda b, j: (b, 0, 0)),
            pl.BlockSpec((1, pb, 3), lambda b, j: (b, j, 0)),
            pl.BlockSpec((1, pb, 9), lambda b, j: (b, j, 0)),
        ],
        out_specs=[
            pl.BlockSpec((pb, 1, n + 16),
                         lambda b, j: (b * (p // pb) + j, 0, 0)),
            pl.BlockSpec((pb, 1, NSLOT),
                         lambda b, j: (b * (p // pb) + j, 0, 0)),
        ],
        out_shape=[
            jax.ShapeDtypeStruct((bsz * p, 1, n + 16), jnp.float32),
            jax.ShapeDtypeStruct((bsz * p, 1, NSLOT), jnp.int32),
        ],
    )(xyz_t, new_xyz, rot9)


def _kminmax(ak, ai, bk, bi):
    sel = ak <= bk
    return (jnp.where(sel, ak, bk), jnp.where(sel, ai, bi),
            jnp.where(sel, bk, ak), jnp.where(sel, bi, ai))


def _bitonic_fix(run):
    n = len(run)
    if n == 1:
        k, i = run[0]
        kk, ii = plsc.sort_key_val(k, i)
        return [(kk, ii)]
    h = n // 2
    lo, hi = [], []
    for j in range(h):
        lk, li, hk, hi_i = _kminmax(run[j][0], run[j][1],
                                    run[j + h][0], run[j + h][1])
        lo.append((lk, li))
        hi.append((hk, hi_i))
    return _bitonic_fix(lo) + _bitonic_fix(hi)


def _merge(a, b, trunc=False):
    n = len(a)
    bp = [(lax.rev(b[n - 1 - j][0], (0,)), lax.rev(b[n - 1 - j][1], (0,)))
          for j in range(n)]
    lo, hi = [], []
    for j in range(n):
        lk, li, hk, hi_i = _kminmax(a[j][0], a[j][1], bp[j][0], bp[j][1])
        lo.append((lk, li))
        hi.append((hk, hi_i))
    if trunc:
        return _bitonic_fix(lo)
    return _bitonic_fix(lo) + _bitonic_fix(hi)


def _sort_lowest64(pairs):
    runs = [[plsc.sort_key_val(k, i)] for k, i in pairs]
    while len(runs) > 1:
        trunc = len(runs[0]) >= 4
        runs = [_merge(runs[2 * j], runs[2 * j + 1], trunc=trunc)
                for j in range(len(runs) // 2)]
    return runs[0]


def _extract(dist, cmeta, xyz_flat, ftab, *, n, c, bsz):
    rows = dist.shape[0]
    rpw = rows // _NW
    mesh = plsc.VectorSubcoreMesh(core_axis_name="c", subcore_axis_name="s")

    def body(dist_hbm, cmeta_hbm, xyzf_hbm, ftab_hbm, fout_hbm, xout_hbm,
             x0_v, x1_v, x2_v, da_v, db_v, svv_v, svi_v, gb_v, fb_v, xo_v,
             cm_s,
             semda, semdb, semf, semwf, semwx):
        wid = lax.axis_index("s") * _NC + lax.axis_index("c")
        b = wid // (_NW // bsz)
        base_row = wid * rpw
        pltpu.async_copy(dist_hbm.at[base_row], da_v, semda)
        pltpu.sync_copy(xyzf_hbm.at[pl.ds((b * 3 + 0) * n, n)], x0_v)
        pltpu.sync_copy(xyzf_hbm.at[pl.ds((b * 3 + 1) * n, n)], x1_v)
        pltpu.sync_copy(xyzf_hbm.at[pl.ds((b * 3 + 2) * n, n)], x2_v)
        iota = lax.iota(jnp.int32, 16)
        cutv = jnp.full((16,), VALID_CUT, jnp.float32)
        capv = jnp.full((16,), CAP, jnp.int32)
        last_row = base_row + rpw - 1

        def process(row, drow_v, dsem, nxt_v, nsem):
            @pl.when(row > base_row)
            def _():
                pltpu.make_async_copy(fb_v, fout_hbm.at[pl.ds(0, S)],
                                      semwf).wait()
                pltpu.make_async_copy(xo_v, xout_hbm.at[pl.ds(0, S)],
                                      semwx).wait()

            pltpu.make_async_copy(dist_hbm.at[base_row], drow_v, dsem).wait()
            nr = jnp.minimum(row + 1, last_row)
            pltpu.async_copy(dist_hbm.at[nr], nxt_v, nsem)
            pltpu.sync_copy(cmeta_hbm.at[row],
                            cm_s.at[pl.ds(0, 1), pl.ds(0, NSLOT)])
            tv = drow_v[0, pl.ds(n, 16)]
            padk = jnp.full((16,), PAD, jnp.float32)
            padi = jnp.zeros((16,), jnp.int32)
            for j in range(CAP // 16 + 1):
                svv_v[pl.ds(j * 16, 16)] = padk
                svi_v[pl.ds(j * 16, 16)] = padi

            hotcnt = cm_s[0, pl.ds(NSLOT - 16, 16)][15]

            def slot(s, _):
                meta = cm_s[0, pl.ds(s, 16)][0]
                ch = meta >> 9
                cb = meta & 511
                off = jnp.full((16,), cb, jnp.int32)
                for u in range(4):
                    lo = ch * 64 + u * 16
                    v = drow_v[0, pl.ds(lo, 16)]
                    m = (v <= tv) & (v < cutv)
                    mi = m.astype(jnp.int32)
                    pos = off + plsc.cumsum(mi) - mi
                    plsc.store_scatter(svv_v, [pos], v, mask=m)
                    plsc.store_scatter(svi_v, [pos], iota + lo, mask=m)
                    cnt = plsc.all_reduce_population_count(m)
                    off = off + cnt
                return 0

            lax.fori_loop(0, hotcnt, slot, 0)

            pairs = [(svv_v[pl.ds(j * 16, 16)], svi_v[pl.ds(j * 16, 16)])
                     for j in range(CAP // 16)]
            low = _sort_lowest64(pairs)
            first = jnp.max(jnp.where(iota == 0, low[0][1], 0))
            fsplat = jnp.full((16,), first, jnp.int32)
            sel = [jnp.where(k < cutv, ii, fsplat) for k, ii in low]
            for j in range(4):
                gb_v[pl.ds(j * 16, 16)] = sel[j] + b * n
            cp = pltpu.async_copy(ftab_hbm.at[gb_v], fb_v, semf)
            zero16 = jnp.zeros((16,), jnp.int32)
            one16 = jnp.full((16,), 1, jnp.int32)
            two16 = jnp.full((16,), 2, jnp.int32)
            for j in range(4):
                rows16 = iota + j * 16
                gx = plsc.load_gather(x0_v, [sel[j]])
                gy = plsc.load_gather(x1_v, [sel[j]])
                gz = plsc.load_gather(x2_v, [sel[j]])
                plsc.store_scatter(xo_v, [rows16, zero16], gx)
                plsc.store_scatter(xo_v, [rows16, one16], gy)
                plsc.store_scatter(xo_v, [rows16, two16], gz)
            cp.wait()
            pltpu.async_copy(fb_v, fout_hbm.at[pl.ds(row * S, S)], semwf)
            pltpu.async_copy(xo_v, xout_hbm.at[pl.ds(row * S, S)], semwx)

        def row_pair(i, _):
            row_a = base_row + 2 * i
            process(row_a, da_v, semda, db_v, semdb)
            process(row_a + 1, db_v, semdb, da_v, semda)
            return 0

        lax.fori_loop(0, rpw // 2, row_pair, 0)
        pltpu.make_async_copy(fb_v, fout_hbm.at[pl.ds(0, S)], semwf).wait()
        pltpu.make_async_copy(xo_v, xout_hbm.at[pl.ds(0, S)], semwx).wait()
        pltpu.make_async_copy(dist_hbm.at[base_row], da_v, semda).wait()

    fn = pl.kernel(
        body,
        out_type=(
            jax.ShapeDtypeStruct((rows * S, c), jnp.float32),
            jax.ShapeDtypeStruct((rows * S, XW), jnp.float32),
        ),
        mesh=mesh,
        compiler_params=pltpu.CompilerParams(needs_layout_passes=False),
        scratch_types=[
            pltpu.VMEM((n,), jnp.float32),
            pltpu.VMEM((n,), jnp.float32),
            pltpu.VMEM((n,), jnp.float32),
            pltpu.VMEM((1, n + 16), jnp.float32),
            pltpu.VMEM((1, n + 16), jnp.float32),
            pltpu.VMEM((SVN,), jnp.float32),
            pltpu.VMEM((SVN,), jnp.int32),
            pltpu.VMEM((S,), jnp.int32),
            pltpu.VMEM((S, c), jnp.float32),
            pltpu.VMEM((S, XW), jnp.float32),
            pltpu.VMEM((1, NSLOT + 16), jnp.int32),
            pltpu.SemaphoreType.DMA,
            pltpu.SemaphoreType.DMA,
            pltpu.SemaphoreType.DMA,
            pltpu.SemaphoreType.DMA,
            pltpu.SemaphoreType.DMA,
        ],
    )
    return fn(dist, cmeta, xyz_flat, ftab)


def _fixup_body(f_ref, x_ref, c_ref, r_ref, out_ref, *, pg, cch):
    w = pg * S
    feats = jnp.transpose(f_ref[0, 0])
    xt = jnp.transpose(x_ref[0, 0])
    lane = lax.broadcasted_iota(jnp.int32, (1, w), 1)
    pidx = lane // S

    def expand(read):
        acc = jnp.full((1, w), read(0), jnp.float32)
        for p in range(1, pg):
            acc = jnp.where(pidx == p, read(p), acc)
        return acc

    bf = lambda a: a.astype(jnp.bfloat16).astype(jnp.float32)
    rel = [bf(xt[d:d + 1, :] - expand(lambda p, d=d: c_ref[0, p, d]))
           for d in range(3)]
    outs = []
    for d in range(3):
        acc = rel[0] * bf(expand(lambda p, d=d: r_ref[0, p, d]))
        acc = acc + rel[1] * bf(expand(lambda p, d=d: r_ref[0, p, 3 + d]))
        acc = acc + rel[2] * bf(expand(lambda p, d=d: r_ref[0, p, 6 + d]))
        outs.append(acc)
    res = jnp.concatenate(outs + [feats], axis=0)
    out_ref[0] = jnp.reshape(res, (3 + cch, pg, S))


def _fixup(frows, xrows, new_xyz, rot9, *, pg=8):
    bsz, p, _ = new_xyz.shape
    cch = frows.shape[-1]
    fr = frows.reshape(bsz, p // pg, pg * S, cch)
    xr = xrows.reshape(bsz, p // pg, pg * S, xrows.shape[-1])
    grid = (bsz, p // pg)
    out = pl.pallas_call(
        functools.partial(_fixup_body, pg=pg, cch=cch),
        grid=grid,
        in_specs=[
            pl.BlockSpec((1, 1, pg * S, cch), lambda b, j: (b, j, 0, 0)),
            pl.BlockSpec((1, 1, pg * S, xr.shape[-1]), lambda b, j: (b, j, 0, 0)),
            pl.BlockSpec((1, pg, 3), lambda b, j: (b, j, 0),
                         memory_space=pltpu.SMEM),
            pl.BlockSpec((1, pg, 9), lambda b, j: (b, j, 0),
                         memory_space=pltpu.SMEM),
        ],
        out_specs=pl.BlockSpec((1, 3 + cch, pg, S), lambda b, j: (b, 0, j, 0)),
        out_shape=jax.ShapeDtypeStruct((bsz, 3 + cch, p, S), jnp.float32),
    )(fr, xr, new_xyz, rot9)
    return out


def kernel(xyz, new_xyz, rot, features):
    bsz, n, _ = xyz.shape
    p = new_xyz.shape[1]
    c = features.shape[1]

    xyz_t = jnp.transpose(xyz, (0, 2, 1))
    rot9 = rot.reshape(bsz, p, 9)

    frs, xrs = [], []
    for bi in range(bsz):
        xt_b = xyz_t[bi:bi + 1]
        d_b, cm_b = _distthr(xt_b, new_xyz[bi:bi + 1], rot9[bi:bi + 1])
        ftab_b = jnp.transpose(features[bi], (1, 0))
        fr, xr = _extract(d_b, cm_b, xt_b.reshape(-1), ftab_b,
                          n=n, c=c, bsz=1)
        frs.append(fr)
        xrs.append(xr)
    frows = jnp.concatenate(frs, axis=0)
    xrows = jnp.concatenate(xrs, axis=0)

    return _fixup(frows, xrows, new_xyz, rot9)
